# Initial kernel scaffold; baseline (speedup 1.0000x reference)
import jax
import jax.numpy as jnp
from jax.experimental import pallas as pl


def kernel(x, pos, edge_index, period_vec, batch, per_config_dataset_idx, elem_table, W_embed, b_embed, W_rbf, W_sh, W_self, W_upd, W_attr, W_p1, b_p1, W_p2, b_p2, scale, shift):
    raise NotImplementedError("write your pallas kernel here")


# speedup vs baseline: 1.2999x; 1.2999x over previous
import functools

import numpy as np
import jax
import jax.numpy as jnp
from jax import lax
from jax.experimental import pallas as pl
from jax.experimental.pallas import tpu as pltpu
from jax.experimental.pallas import tpu_sc as plsc

_NC = 2
_NS = 16
_NW = _NC * _NS

_BE = 2000
_BN = 1000
_CH = 128


def _dsilu(x):
    s = jax.nn.sigmoid(x)
    return s * (1.0 + x * (1.0 - s))


@functools.lru_cache(maxsize=None)
def _sc_gather_fn(N, H, E):
    nch = E // _CH
    assert nch * _CH == E
    per_w = -(-nch // _NW)
    mesh = plsc.VectorSubcoreMesh(core_axis_name="c", subcore_axis_name="s",
                                  num_cores=_NC, num_subcores=_NS)

    @functools.partial(
        pl.kernel, mesh=mesh,
        out_type=jax.ShapeDtypeStruct((E, H), jnp.float32),
        scratch_types=[
            pltpu.VMEM((_CH,), jnp.int32),
            pltpu.VMEM((_CH, H), jnp.float32),
            pltpu.SemaphoreType.DMA,
        ],
    )
    def k(table_hbm, idx_hbm, out_hbm, idx_v, rows_v, sem):
        wid = lax.axis_index("s") * _NC + lax.axis_index("c")

        def body(i, _):
            j = i * _NW + wid

            @pl.when(j < nch)
            def _():
                base = j * _CH
                pltpu.sync_copy(idx_hbm.at[pl.ds(base, _CH)], idx_v)
                pltpu.async_copy(table_hbm.at[idx_v], rows_v, sem).wait()
                pltpu.sync_copy(rows_v, out_hbm.at[pl.ds(base, _CH)])
            return 0

        lax.fori_loop(0, per_w, body, 0)

    return k


@functools.lru_cache(maxsize=None)
def _sc_scatter_fn(N, H, E):
    Hc = H // _NC
    nch = E // _CH
    assert nch * _CH == E
    per_s = -(-nch // _NS)
    zch = 80
    nz = N // zch
    assert nz * zch == N
    per_sz = -(-nz // _NS)
    mesh = plsc.VectorSubcoreMesh(core_axis_name="c", subcore_axis_name="s",
                                  num_cores=_NC, num_subcores=_NS)

    @functools.partial(
        pl.kernel, mesh=mesh,
        out_type=jax.ShapeDtypeStruct((N, H), jnp.float32),
        scratch_types=[
            pltpu.VMEM((_CH,), jnp.int32),
            pltpu.VMEM((_CH, Hc), jnp.float32),
            pltpu.VMEM((zch, Hc), jnp.float32),
            pltpu.VMEM_SHARED((N, Hc), jnp.float32),
            pltpu.SemaphoreType.DMA,
        ],
    )
    def k(msg_hbm, idx_hbm, out_hbm, idx_v, rows_v, stage_v, acc_sh, sem):
        c = lax.axis_index("c")
        s = lax.axis_index("s")

        def zrow(i, _):
            def zcol(j, _):
                stage_v[i, pl.ds(j * 16, 16)] = jnp.zeros((16,), jnp.float32)
                return 0
            lax.fori_loop(0, Hc // 16, zcol, 0)
            return 0
        lax.fori_loop(0, zch, zrow, 0)

        def zdma(k, _):
            t = k * _NS + s

            @pl.when(t < nz)
            def _():
                pltpu.sync_copy(stage_v, acc_sh.at[pl.ds(t * zch, zch)])
            return 0
        lax.fori_loop(0, per_sz, zdma, 0)
        plsc.subcore_barrier()

        def body(i, _):
            j = i * _NS + s

            @pl.when(j < nch)
            def _():
                base = j * _CH
                pltpu.sync_copy(idx_hbm.at[pl.ds(base, _CH)], idx_v)

                @pl.when(c == 0)
                def _():
                    pltpu.sync_copy(
                        msg_hbm.at[pl.ds(base, _CH), pl.ds(0, Hc)], rows_v)

                @pl.when(c == 1)
                def _():
                    pltpu.sync_copy(
                        msg_hbm.at[pl.ds(base, _CH), pl.ds(Hc, Hc)], rows_v)

                pltpu.sync_copy(rows_v, acc_sh.at[idx_v], add=True)
            return 0

        lax.fori_loop(0, per_s, body, 0)
        plsc.subcore_barrier()

        def wb(k, _):
            t = k * _NS + s

            @pl.when(t < nz)
            def _():
                r0 = t * zch
                pltpu.sync_copy(acc_sh.at[pl.ds(r0, zch)], stage_v)

                @pl.when(c == 0)
                def _():
                    pltpu.sync_copy(stage_v,
                                    out_hbm.at[pl.ds(r0, zch), pl.ds(0, Hc)])

                @pl.when(c == 1)
                def _():
                    pltpu.sync_copy(stage_v,
                                    out_hbm.at[pl.ds(r0, zch), pl.ds(Hc, Hc)])
            return 0

        lax.fori_loop(0, per_sz, wb, 0)

    return k


def _gather_call(table, idx):
    N, H = table.shape
    (E,) = idx.shape
    return _sc_gather_fn(N, H, E)(table, idx)


def _scatter_call(msg, idx, N):
    E, H = msg.shape
    return _sc_scatter_fn(N, H, E)(msg, idx)


def _f0(idx):
    return np.zeros(idx.shape, jax.dtypes.float0)


def _make_sc_ops(N):
    @jax.custom_vjp
    def gather_rows(table, idx):
        return _gather_call(table, idx)

    def gather_fwd(table, idx):
        return _gather_call(table, idx), idx

    def gather_bwd(idx, g):
        return _scatter_call(g, idx, N), _f0(idx)

    gather_rows.defvjp(gather_fwd, gather_bwd)

    @jax.custom_vjp
    def seg_sum(msg, idx):
        return _scatter_call(msg, idx, N)

    def seg_fwd(msg, idx):
        return _scatter_call(msg, idx, N), idx

    def seg_bwd(idx, g):
        return _gather_call(g, idx), _f0(idx)

    seg_sum.defvjp(seg_fwd, seg_bwd)
    return gather_rows, seg_sum


def _mm(a, b):
    return jnp.dot(a, b, preferred_element_type=jnp.float32)


@functools.lru_cache(maxsize=None)
def _edge_mod_fwd_fn(E, H, K1, K2):
    def body(rbf_ref, shp_ref, wr_ref, wsp_ref, mod_ref):
        radial = _mm(rbf_ref[...], wr_ref[...])
        shw = _mm(shp_ref[...], wsp_ref[...])
        mod_ref[...] = radial * shw

    return pl.pallas_call(
        body,
        grid=(E // _BE,),
        in_specs=[
            pl.BlockSpec((_BE, K1), lambda i: (i, 0)),
            pl.BlockSpec((_BE, K2), lambda i: (i, 0)),
            pl.BlockSpec((K1, H), lambda i: (0, 0)),
            pl.BlockSpec((K2, H), lambda i: (0, 0)),
        ],
        out_specs=pl.BlockSpec((_BE, H), lambda i: (i, 0)),
        out_shape=jax.ShapeDtypeStruct((E, H), jnp.float32),
    )


@functools.lru_cache(maxsize=None)
def _edge_mod_bwd_fn(E, H, K1, K2):
    def body(g_ref, rbf_ref, shp_ref, wr_ref, wsp_ref, wrt_ref, wspt_ref,
             drbf_ref, dshp_ref):
        g = g_ref[...]
        radial = _mm(rbf_ref[...], wr_ref[...])
        shw = _mm(shp_ref[...], wsp_ref[...])
        drbf_ref[...] = _mm(g * shw, wrt_ref[...])
        dshp_ref[...] = _mm(g * radial, wspt_ref[...])

    return pl.pallas_call(
        body,
        grid=(E // _BE,),
        in_specs=[
            pl.BlockSpec((_BE, H), lambda i: (i, 0)),
            pl.BlockSpec((_BE, K1), lambda i: (i, 0)),
            pl.BlockSpec((_BE, K2), lambda i: (i, 0)),
            pl.BlockSpec((K1, H), lambda i: (0, 0)),
            pl.BlockSpec((K2, H), lambda i: (0, 0)),
            pl.BlockSpec((H, K1), lambda i: (0, 0)),
            pl.BlockSpec((H, K2), lambda i: (0, 0)),
        ],
        out_specs=[
            pl.BlockSpec((_BE, K1), lambda i: (i, 0)),
            pl.BlockSpec((_BE, K2), lambda i: (i, 0)),
        ],
        out_shape=[
            jax.ShapeDtypeStruct((E, K1), jnp.float32),
            jax.ShapeDtypeStruct((E, K2), jnp.float32),
        ],
    )


def _edge_mod_fwd_call(rbf, shp, wr, wsp):
    E, K1 = rbf.shape
    K2 = shp.shape[1]
    H = wr.shape[1]
    return _edge_mod_fwd_fn(E, H, K1, K2)(rbf, shp, wr, wsp)


def _edge_mod_bwd_call(g, rbf, shp, wr, wsp):
    E, K1 = rbf.shape
    K2 = shp.shape[1]
    H = wr.shape[1]
    return _edge_mod_bwd_fn(E, H, K1, K2)(g, rbf, shp, wr, wsp, wr.T, wsp.T)


@jax.custom_vjp
def _edge_mod(rbf, shp, wr, wsp):
    return _edge_mod_fwd_call(rbf, shp, wr, wsp)


def _edge_mod_f(rbf, shp, wr, wsp):
    return _edge_mod_fwd_call(rbf, shp, wr, wsp), (rbf, shp, wr, wsp)


def _edge_mod_b(res, g):
    rbf, shp, wr, wsp = res
    drbf, dshp = _edge_mod_bwd_call(g, rbf, shp, wr, wsp)
    return drbf, dshp, jnp.zeros_like(wr), jnp.zeros_like(wsp)


_edge_mod.defvjp(_edge_mod_f, _edge_mod_b)


@functools.lru_cache(maxsize=None)
def _node_upd_fwd_fn(N, H, A):
    def body(h_ref, agg_ref, xa_ref, ws_ref, wu_ref, wa_ref, pre_ref, hn_ref):
        pre = (_mm(h_ref[...], ws_ref[...]) + _mm(agg_ref[...], wu_ref[...])
               + _mm(xa_ref[...], wa_ref[...]))
        pre_ref[...] = pre
        hn_ref[...] = pre * jax.nn.sigmoid(pre)

    return pl.pallas_call(
        body,
        grid=(N // _BN,),
        in_specs=[
            pl.BlockSpec((_BN, H), lambda i: (i, 0)),
            pl.BlockSpec((_BN, H), lambda i: (i, 0)),
            pl.BlockSpec((_BN, A), lambda i: (i, 0)),
            pl.BlockSpec((H, H), lambda i: (0, 0)),
            pl.BlockSpec((H, H), lambda i: (0, 0)),
            pl.BlockSpec((A, H), lambda i: (0, 0)),
        ],
        out_specs=[
            pl.BlockSpec((_BN, H), lambda i: (i, 0)),
            pl.BlockSpec((_BN, H), lambda i: (i, 0)),
        ],
        out_shape=[
            jax.ShapeDtypeStruct((N, H), jnp.float32),
            jax.ShapeDtypeStruct((N, H), jnp.float32),
        ],
    )


@functools.lru_cache(maxsize=None)
def _node_upd_bwd_fn(N, H):
    def body(g_ref, pre_ref, wst_ref, wut_ref, dh_ref, dagg_ref):
        gp = g_ref[...] * _dsilu(pre_ref[...])
        dh_ref[...] = _mm(gp, wst_ref[...])
        dagg_ref[...] = _mm(gp, wut_ref[...])

    return pl.pallas_call(
        body,
        grid=(N // _BN,),
        in_specs=[
            pl.BlockSpec((_BN, H), lambda i: (i, 0)),
            pl.BlockSpec((_BN, H), lambda i: (i, 0)),
            pl.BlockSpec((H, H), lambda i: (0, 0)),
            pl.BlockSpec((H, H), lambda i: (0, 0)),
        ],
        out_specs=[
            pl.BlockSpec((_BN, H), lambda i: (i, 0)),
            pl.BlockSpec((_BN, H), lambda i: (i, 0)),
        ],
        out_shape=[
            jax.ShapeDtypeStruct((N, H), jnp.float32),
            jax.ShapeDtypeStruct((N, H), jnp.float32),
        ],
    )


def _node_upd_fwd_call(h, agg, xa, ws, wu, wa):
    N, H = h.shape
    A = xa.shape[1]
    return _node_upd_fwd_fn(N, H, A)(h, agg, xa, ws, wu, wa)


def _node_upd_bwd_call(g, pre, ws, wu):
    N, H = g.shape
    return _node_upd_bwd_fn(N, H)(g, pre, ws.T, wu.T)


@jax.custom_vjp
def _node_update(h, agg, xa, ws, wu, wa):
    _, hn = _node_upd_fwd_call(h, agg, xa, ws, wu, wa)
    return hn


def _node_update_f(h, agg, xa, ws, wu, wa):
    pre, hn = _node_upd_fwd_call(h, agg, xa, ws, wu, wa)
    return hn, (pre, ws, wu, xa.shape[1])


def _node_update_b(res, g):
    pre, ws, wu, A = res
    dh, dagg = _node_upd_bwd_call(g, pre, ws, wu)
    return (dh, dagg, jnp.zeros((g.shape[0], A), jnp.float32),
            jnp.zeros_like(ws), jnp.zeros_like(wu),
            jnp.zeros((A, g.shape[1]), jnp.float32))


_node_update.defvjp(_node_update_f, _node_update_b)


@functools.lru_cache(maxsize=None)
def _head_fwd_fn(N, H, P):
    def body(h_ref, w_ref, b_ref, z_ref, hp_ref):
        z = _mm(h_ref[...], w_ref[...]) + b_ref[...]
        z_ref[...] = z
        hp_ref[...] = z * jax.nn.sigmoid(z)

    return pl.pallas_call(
        body,
        grid=(N // _BN,),
        in_specs=[
            pl.BlockSpec((_BN, H), lambda i: (i, 0)),
            pl.BlockSpec((H, P), lambda i: (0, 0)),
            pl.BlockSpec((1, P), lambda i: (0, 0)),
        ],
        out_specs=[
            pl.BlockSpec((_BN, P), lambda i: (i, 0)),
            pl.BlockSpec((_BN, P), lambda i: (i, 0)),
        ],
        out_shape=[
            jax.ShapeDtypeStruct((N, P), jnp.float32),
            jax.ShapeDtypeStruct((N, P), jnp.float32),
        ],
    )


@functools.lru_cache(maxsize=None)
def _head_bwd_fn(N, H, P):
    def body(g_ref, z_ref, wt_ref, dh_ref):
        dh_ref[...] = _mm(g_ref[...] * _dsilu(z_ref[...]), wt_ref[...])

    return pl.pallas_call(
        body,
        grid=(N // _BN,),
        in_specs=[
            pl.BlockSpec((_BN, P), lambda i: (i, 0)),
            pl.BlockSpec((_BN, P), lambda i: (i, 0)),
            pl.BlockSpec((P, H), lambda i: (0, 0)),
        ],
        out_specs=pl.BlockSpec((_BN, H), lambda i: (i, 0)),
        out_shape=jax.ShapeDtypeStruct((N, H), jnp.float32),
    )


def _head_fwd_call(h, w, b):
    N, H = h.shape
    P = w.shape[1]
    return _head_fwd_fn(N, H, P)(h, w, b.reshape(1, P))


def _head_bwd_call(g, z, w):
    N, P = g.shape
    H = w.shape[0]
    return _head_bwd_fn(N, H, P)(g, z, w.T)


@jax.custom_vjp
def _head(h, w, b):
    _, hp = _head_fwd_call(h, w, b)
    return hp


def _head_f(h, w, b):
    z, hp = _head_fwd_call(h, w, b)
    return hp, (z, w)


def _head_b(res, g):
    z, w = res
    return _head_bwd_call(g, z, w), jnp.zeros_like(w), jnp.zeros((w.shape[1],), jnp.float32)


_head.defvjp(_head_f, _head_b)


@functools.lru_cache(maxsize=None)
def _embed_fn(N, A, H):
    def body(xa_ref, w_ref, b_ref, out_ref):
        out_ref[...] = _mm(xa_ref[...], w_ref[...]) + b_ref[...]

    return pl.pallas_call(
        body,
        grid=(N // _BN,),
        in_specs=[
            pl.BlockSpec((_BN, A), lambda i: (i, 0)),
            pl.BlockSpec((A, H), lambda i: (0, 0)),
            pl.BlockSpec((1, H), lambda i: (0, 0)),
        ],
        out_specs=pl.BlockSpec((_BN, H), lambda i: (i, 0)),
        out_shape=jax.ShapeDtypeStruct((N, H), jnp.float32),
    )


def _embed_call(xa, w, b):
    N, A = xa.shape
    H = w.shape[1]
    return _embed_fn(N, A, H)(xa, w, b.reshape(1, H))


def kernel(x, pos, edge_index, period_vec, batch, per_config_dataset_idx,
           elem_table, W_embed, b_embed, W_rbf, W_sh, W_self, W_upd, W_attr,
           W_p1, b_p1, W_p2, b_p2, scale, shift):
    N = pos.shape[0]
    E = edge_index.shape[1]
    H = W_embed.shape[1]
    G = per_config_dataset_idx.shape[0]
    nlayers, nrbf, _ = W_rbf.shape
    sh_dim = W_sh.shape[1]
    cutoff = 6.0

    src = edge_index[0]
    dst = edge_index[1]
    x_attr = elem_table[x]
    h0 = _embed_call(x_attr, W_embed, b_embed)
    k2 = 16
    W_shp = jnp.concatenate(
        [W_sh, jnp.zeros((nlayers, k2 - sh_dim, H), jnp.float32)], axis=1)

    gather_rows, seg_sum = _make_sc_ops(N)

    centers = jnp.linspace(0.0, cutoff, nrbf)
    width = cutoff / nrbf

    def efn(pos_in):
        edge_vec = pos_in[dst] - pos_in[src] + period_vec
        lengths = jnp.sqrt(jnp.sum(edge_vec * edge_vec, axis=-1) + 1e-12)
        unit = edge_vec / lengths[:, None]
        ex, ey, ez = unit[:, 0], unit[:, 1], unit[:, 2]
        zero = jnp.zeros_like(ex)
        shp = jnp.stack(
            [jnp.ones_like(ex), ex, ey, ez, ex * ey, ey * ez, ez * ex,
             ex * ex - ey * ey, 3.0 * ez * ez - 1.0,
             zero, zero, zero, zero, zero, zero, zero], axis=-1)
        rbf = jnp.exp(-jnp.square((lengths[:, None] - centers[None, :]) / width))
        env = 0.5 * (jnp.cos(jnp.pi * jnp.clip(lengths / cutoff, 0.0, 1.0)) + 1.0)
        rbf = rbf * env[:, None]

        h = h0
        for l in range(nlayers):
            mod = _edge_mod(rbf, shp, W_rbf[l], W_shp[l])
            gath = gather_rows(h, src)
            msg = gath * mod
            agg = seg_sum(msg, dst)
            h = _node_update(h, agg, x_attr, W_self[l], W_upd[l], W_attr[l])

        hp1 = _head(h, W_p1, b_p1)
        hp2 = _mm(hp1, W_p2) + b_p2
        graph_e = jax.ops.segment_sum(hp2, batch, num_segments=G)
        energies_all = graph_e * scale + shift
        return energies_all[jnp.arange(G), per_config_dataset_idx]

    energies, vjp_fn = jax.vjp(efn, pos)
    forces = -vjp_fn(jnp.ones_like(energies))[0]
    return (energies, forces)


# speedup vs baseline: 1.4051x; 1.0809x over previous
import functools

import numpy as np
import jax
import jax.numpy as jnp
from jax import lax
from jax.experimental import pallas as pl
from jax.experimental.pallas import tpu as pltpu
from jax.experimental.pallas import tpu_sc as plsc

_NC = 2
_NS = 16
_NW = _NC * _NS

_BE = 2000
_BN = 1000
_CH = 128


def _dsilu(x):
    s = jax.nn.sigmoid(x)
    return s * (1.0 + x * (1.0 - s))


@functools.lru_cache(maxsize=None)
def _sc_gather_fn(N, H, E):
    nch = E // _CH
    assert nch * _CH == E
    per_w = -(-nch // _NW)
    mesh = plsc.VectorSubcoreMesh(core_axis_name="c", subcore_axis_name="s",
                                  num_cores=_NC, num_subcores=_NS)

    @functools.partial(
        pl.kernel, mesh=mesh,
        out_type=jax.ShapeDtypeStruct((E, H), jnp.float32),
        scratch_types=[
            pltpu.VMEM((_CH,), jnp.int32),
            pltpu.VMEM((_CH, H), jnp.float32),
            pltpu.SemaphoreType.DMA,
        ],
    )
    def k(table_hbm, idx_hbm, out_hbm, idx_v, rows_v, sem):
        wid = lax.axis_index("s") * _NC + lax.axis_index("c")

        def body(i, _):
            j = i * _NW + wid

            @pl.when(j < nch)
            def _():
                base = j * _CH
                pltpu.sync_copy(idx_hbm.at[pl.ds(base, _CH)], idx_v)
                pltpu.async_copy(table_hbm.at[idx_v], rows_v, sem).wait()
                pltpu.sync_copy(rows_v, out_hbm.at[pl.ds(base, _CH)])
            return 0

        lax.fori_loop(0, per_w, body, 0)

    return k


@functools.lru_cache(maxsize=None)
def _sc_scatter_fn(N, H, E):
    Hc = H // _NC
    nch = E // _CH
    assert nch * _CH == E
    per_s = -(-nch // _NS)
    zch = 80
    nz = N // zch
    assert nz * zch == N
    per_sz = -(-nz // _NS)
    mesh = plsc.VectorSubcoreMesh(core_axis_name="c", subcore_axis_name="s",
                                  num_cores=_NC, num_subcores=_NS)

    @functools.partial(
        pl.kernel, mesh=mesh,
        out_type=jax.ShapeDtypeStruct((N, H), jnp.float32),
        scratch_types=[
            pltpu.VMEM((_CH,), jnp.int32),
            pltpu.VMEM((_CH, Hc), jnp.float32),
            pltpu.VMEM((zch, Hc), jnp.float32),
            pltpu.VMEM_SHARED((N, Hc), jnp.float32),
            pltpu.SemaphoreType.DMA,
        ],
    )
    def k(msg_hbm, idx_hbm, out_hbm, idx_v, rows_v, stage_v, acc_sh, sem):
        c = lax.axis_index("c")
        s = lax.axis_index("s")

        def zrow(i, _):
            def zcol(j, _):
                stage_v[i, pl.ds(j * 16, 16)] = jnp.zeros((16,), jnp.float32)
                return 0
            lax.fori_loop(0, Hc // 16, zcol, 0)
            return 0
        lax.fori_loop(0, zch, zrow, 0)

        def zdma(k, _):
            t = k * _NS + s

            @pl.when(t < nz)
            def _():
                pltpu.sync_copy(stage_v, acc_sh.at[pl.ds(t * zch, zch)])
            return 0
        lax.fori_loop(0, per_sz, zdma, 0)
        plsc.subcore_barrier()

        def body(i, _):
            j = i * _NS + s

            @pl.when(j < nch)
            def _():
                base = j * _CH
                pltpu.sync_copy(idx_hbm.at[pl.ds(base, _CH)], idx_v)

                @pl.when(c == 0)
                def _():
                    pltpu.sync_copy(
                        msg_hbm.at[pl.ds(base, _CH), pl.ds(0, Hc)], rows_v)

                @pl.when(c == 1)
                def _():
                    pltpu.sync_copy(
                        msg_hbm.at[pl.ds(base, _CH), pl.ds(Hc, Hc)], rows_v)

                pltpu.sync_copy(rows_v, acc_sh.at[idx_v], add=True)
            return 0

        lax.fori_loop(0, per_s, body, 0)
        plsc.subcore_barrier()

        def wb(k, _):
            t = k * _NS + s

            @pl.when(t < nz)
            def _():
                r0 = t * zch
                pltpu.sync_copy(acc_sh.at[pl.ds(r0, zch)], stage_v)

                @pl.when(c == 0)
                def _():
                    pltpu.sync_copy(stage_v,
                                    out_hbm.at[pl.ds(r0, zch), pl.ds(0, Hc)])

                @pl.when(c == 1)
                def _():
                    pltpu.sync_copy(stage_v,
                                    out_hbm.at[pl.ds(r0, zch), pl.ds(Hc, Hc)])
            return 0

        lax.fori_loop(0, per_sz, wb, 0)

    return k


@functools.lru_cache(maxsize=None)
def _sc_scatter_pad_fn(N, E2, W):
    nch = E2 // _CH
    assert nch * _CH == E2
    nch_c = -(-nch // _NC)
    per_s = -(-nch_c // _NS)
    zch = 80
    nz = N // zch
    assert nz * zch == N
    per_sz = -(-nz // _NS)
    mesh = plsc.VectorSubcoreMesh(core_axis_name="c", subcore_axis_name="s",
                                  num_cores=_NC, num_subcores=_NS)

    @functools.partial(
        pl.kernel, mesh=mesh,
        out_type=jax.ShapeDtypeStruct((2, N, W), jnp.float32),
        scratch_types=[
            pltpu.VMEM((_CH,), jnp.int32),
            pltpu.VMEM((_CH, W), jnp.float32),
            pltpu.VMEM((zch, W), jnp.float32),
            pltpu.VMEM_SHARED((N, W), jnp.float32),
            pltpu.SemaphoreType.DMA,
        ],
    )
    def k(val_hbm, idx_hbm, out_hbm, idx_v, rows_v, stage_v, acc_sh, sem):
        c = lax.axis_index("c")
        s = lax.axis_index("s")

        def zrow(i, _):
            def zcol(j, _):
                stage_v[i, pl.ds(j * 16, 16)] = jnp.zeros((16,), jnp.float32)
                return 0
            lax.fori_loop(0, W // 16, zcol, 0)
            return 0
        lax.fori_loop(0, zch, zrow, 0)

        def zdma(k2, _):
            t = k2 * _NS + s

            @pl.when(t < nz)
            def _():
                pltpu.sync_copy(stage_v, acc_sh.at[pl.ds(t * zch, zch)])
            return 0
        lax.fori_loop(0, per_sz, zdma, 0)
        plsc.subcore_barrier()

        def body(i, _):
            kk = i * _NS + s
            j = kk * _NC + c

            @pl.when(j < nch)
            def _():
                base = j * _CH
                pltpu.sync_copy(idx_hbm.at[pl.ds(base, _CH)], idx_v)
                pltpu.sync_copy(val_hbm.at[pl.ds(base, _CH)], rows_v)
                pltpu.sync_copy(rows_v, acc_sh.at[idx_v], add=True)
            return 0

        lax.fori_loop(0, per_s, body, 0)
        plsc.subcore_barrier()

        def wb(k2, _):
            t = k2 * _NS + s

            @pl.when(t < nz)
            def _():
                r0 = t * zch
                pltpu.sync_copy(acc_sh.at[pl.ds(r0, zch)], stage_v)

                @pl.when(c == 0)
                def _():
                    pltpu.sync_copy(stage_v, out_hbm.at[0, pl.ds(r0, zch)])

                @pl.when(c == 1)
                def _():
                    pltpu.sync_copy(stage_v, out_hbm.at[1, pl.ds(r0, zch)])
            return 0

        lax.fori_loop(0, per_sz, wb, 0)

    return k


def _scatter_pad_call(val, idx, N):
    E2, W = val.shape
    parts = _sc_scatter_pad_fn(N, E2, W)(val, idx)
    return parts[0] + parts[1]


def _make_edge_diff(N, E):
    @jax.custom_vjp
    def edge_diff(pos, period_vec, src, dst):
        return pos[dst] - pos[src] + period_vec

    def ed_fwd(pos, period_vec, src, dst):
        return pos[dst] - pos[src] + period_vec, (src, dst)

    def ed_bwd(res, g):
        src, dst = res
        gp = jnp.pad(g, ((0, 0), (0, 125)))
        val = jnp.concatenate([gp, -gp], axis=0)
        idx = jnp.concatenate([dst, src], axis=0)
        d_pos = _scatter_pad_call(val, idx, N)[:, :3]
        return d_pos, g, _f0(src), _f0(dst)

    edge_diff.defvjp(ed_fwd, ed_bwd)
    return edge_diff


def _vmul_rows(dst_ref, a_ref, b_ref, rows, cols):
    def row(e, _):
        for jj in range(cols // 16):
            sl = pl.ds(jj * 16, 16)
            dst_ref[e, sl] = a_ref[e, sl] * b_ref[e, sl]
        return 0
    lax.fori_loop(0, rows, row, 0)


@functools.lru_cache(maxsize=None)
def _sc_conv_fwd_fn(N, H, E):
    Hc = H // _NC
    nch = E // _CH
    assert nch * _CH == E
    per_s = -(-nch // _NS)
    zch = 80
    nz = N // zch
    assert nz * zch == N
    per_sz = -(-nz // _NS)
    mesh = plsc.VectorSubcoreMesh(core_axis_name="c", subcore_axis_name="s",
                                  num_cores=_NC, num_subcores=_NS)

    @functools.partial(
        pl.kernel, mesh=mesh,
        out_type=jax.ShapeDtypeStruct((N, H), jnp.float32),
        scratch_types=[
            pltpu.VMEM((_CH,), jnp.int32),
            pltpu.VMEM((_CH,), jnp.int32),
            pltpu.VMEM((_CH, Hc), jnp.float32),
            pltpu.VMEM((_CH, Hc), jnp.float32),
            pltpu.VMEM((zch, Hc), jnp.float32),
            pltpu.VMEM_SHARED((N, Hc), jnp.float32),
            pltpu.SemaphoreType.DMA,
        ],
    )
    def k(h0_hbm, h1_hbm, mod_hbm, src_hbm, dst_hbm, out_hbm,
          si_v, di_v, g_v, m_v, stage_v, acc_sh, sem):
        c = lax.axis_index("c")
        s = lax.axis_index("s")

        def zrow(i, _):
            def zcol(j, _):
                stage_v[i, pl.ds(j * 16, 16)] = jnp.zeros((16,), jnp.float32)
                return 0
            lax.fori_loop(0, Hc // 16, zcol, 0)
            return 0
        lax.fori_loop(0, zch, zrow, 0)

        def zdma(k2, _):
            t = k2 * _NS + s

            @pl.when(t < nz)
            def _():
                pltpu.sync_copy(stage_v, acc_sh.at[pl.ds(t * zch, zch)])
            return 0
        lax.fori_loop(0, per_sz, zdma, 0)
        plsc.subcore_barrier()

        def body(i, _):
            j = i * _NS + s

            @pl.when(j < nch)
            def _():
                base = j * _CH
                pltpu.sync_copy(src_hbm.at[pl.ds(base, _CH)], si_v)
                pltpu.sync_copy(dst_hbm.at[pl.ds(base, _CH)], di_v)

                @pl.when(c == 0)
                def _():
                    pltpu.async_copy(h0_hbm.at[si_v], g_v, sem).wait()
                    pltpu.sync_copy(
                        mod_hbm.at[pl.ds(base, _CH), pl.ds(0, Hc)], m_v)

                @pl.when(c == 1)
                def _():
                    pltpu.async_copy(h1_hbm.at[si_v], g_v, sem).wait()
                    pltpu.sync_copy(
                        mod_hbm.at[pl.ds(base, _CH), pl.ds(Hc, Hc)], m_v)

                _vmul_rows(m_v, m_v, g_v, _CH, Hc)
                pltpu.sync_copy(m_v, acc_sh.at[di_v], add=True)
            return 0

        lax.fori_loop(0, per_s, body, 0)
        plsc.subcore_barrier()

        def wb(k2, _):
            t = k2 * _NS + s

            @pl.when(t < nz)
            def _():
                r0 = t * zch
                pltpu.sync_copy(acc_sh.at[pl.ds(r0, zch)], stage_v)

                @pl.when(c == 0)
                def _():
                    pltpu.sync_copy(stage_v,
                                    out_hbm.at[pl.ds(r0, zch), pl.ds(0, Hc)])

                @pl.when(c == 1)
                def _():
                    pltpu.sync_copy(stage_v,
                                    out_hbm.at[pl.ds(r0, zch), pl.ds(Hc, Hc)])
            return 0

        lax.fori_loop(0, per_sz, wb, 0)

    return k


@functools.lru_cache(maxsize=None)
def _sc_conv_bwd_fn(N, H, E):
    Hc = H // _NC
    nch = E // _CH
    assert nch * _CH == E
    per_s = -(-nch // _NS)
    zch = 80
    nz = N // zch
    assert nz * zch == N
    per_sz = -(-nz // _NS)
    mesh = plsc.VectorSubcoreMesh(core_axis_name="c", subcore_axis_name="s",
                                  num_cores=_NC, num_subcores=_NS)

    @functools.partial(
        pl.kernel, mesh=mesh,
        out_type=jax.ShapeDtypeStruct((E, H), jnp.float32),
        scratch_types=[
            pltpu.VMEM((_CH,), jnp.int32),
            pltpu.VMEM((_CH,), jnp.int32),
            pltpu.VMEM((_CH, Hc), jnp.float32),
            pltpu.VMEM((_CH, Hc), jnp.float32),
            pltpu.SemaphoreType.DMA,
            pltpu.SemaphoreType.DMA,
        ],
    )
    def k_dmod(da0_hbm, da1_hbm, h0_hbm, h1_hbm, src_hbm, dst_hbm,
               dmod_hbm, si_v, di_v, t_v, g_v, sem1, sem2):
        c = lax.axis_index("c")
        s = lax.axis_index("s")

        def body(i, _):
            j = i * _NS + s

            @pl.when(j < nch)
            def _():
                base = j * _CH
                pltpu.sync_copy(src_hbm.at[pl.ds(base, _CH)], si_v)
                pltpu.sync_copy(dst_hbm.at[pl.ds(base, _CH)], di_v)

                @pl.when(c == 0)
                def _():
                    cp1 = pltpu.async_copy(da0_hbm.at[di_v], t_v, sem1)
                    cp2 = pltpu.async_copy(h0_hbm.at[si_v], g_v, sem2)
                    cp1.wait()
                    cp2.wait()
                    _vmul_rows(g_v, g_v, t_v, _CH, Hc)
                    pltpu.sync_copy(
                        g_v, dmod_hbm.at[pl.ds(base, _CH), pl.ds(0, Hc)])

                @pl.when(c == 1)
                def _():
                    cp1 = pltpu.async_copy(da1_hbm.at[di_v], t_v, sem1)
                    cp2 = pltpu.async_copy(h1_hbm.at[si_v], g_v, sem2)
                    cp1.wait()
                    cp2.wait()
                    _vmul_rows(g_v, g_v, t_v, _CH, Hc)
                    pltpu.sync_copy(
                        g_v, dmod_hbm.at[pl.ds(base, _CH), pl.ds(Hc, Hc)])
            return 0

        lax.fori_loop(0, per_s, body, 0)

    @functools.partial(
        pl.kernel, mesh=mesh,
        out_type=jax.ShapeDtypeStruct((N, H), jnp.float32),
        scratch_types=[
            pltpu.VMEM((_CH,), jnp.int32),
            pltpu.VMEM((_CH,), jnp.int32),
            pltpu.VMEM((_CH, Hc), jnp.float32),
            pltpu.VMEM((_CH, Hc), jnp.float32),
            pltpu.VMEM((zch, Hc), jnp.float32),
            pltpu.VMEM_SHARED((N, Hc), jnp.float32),
            pltpu.SemaphoreType.DMA,
        ],
    )
    def k_dh(da0_hbm, da1_hbm, mod_hbm, src_hbm, dst_hbm, dh_hbm,
             si_v, di_v, t_v, m_v, stage_v, acc_sh, sem1):
        c = lax.axis_index("c")
        s = lax.axis_index("s")

        def zrow(i, _):
            def zcol(j, _):
                stage_v[i, pl.ds(j * 16, 16)] = jnp.zeros((16,), jnp.float32)
                return 0
            lax.fori_loop(0, Hc // 16, zcol, 0)
            return 0
        lax.fori_loop(0, zch, zrow, 0)

        def zdma(k2, _):
            t = k2 * _NS + s

            @pl.when(t < nz)
            def _():
                pltpu.sync_copy(stage_v, acc_sh.at[pl.ds(t * zch, zch)])
            return 0
        lax.fori_loop(0, per_sz, zdma, 0)
        plsc.subcore_barrier()

        def body(i, _):
            j = i * _NS + s

            @pl.when(j < nch)
            def _():
                base = j * _CH
                pltpu.sync_copy(src_hbm.at[pl.ds(base, _CH)], si_v)
                pltpu.sync_copy(dst_hbm.at[pl.ds(base, _CH)], di_v)

                @pl.when(c == 0)
                def _():
                    cp1 = pltpu.async_copy(da0_hbm.at[di_v], t_v, sem1)
                    pltpu.sync_copy(
                        mod_hbm.at[pl.ds(base, _CH), pl.ds(0, Hc)], m_v)
                    cp1.wait()

                @pl.when(c == 1)
                def _():
                    cp1 = pltpu.async_copy(da1_hbm.at[di_v], t_v, sem1)
                    pltpu.sync_copy(
                        mod_hbm.at[pl.ds(base, _CH), pl.ds(Hc, Hc)], m_v)
                    cp1.wait()

                _vmul_rows(m_v, m_v, t_v, _CH, Hc)
                pltpu.sync_copy(m_v, acc_sh.at[si_v], add=True)
            return 0

        lax.fori_loop(0, per_s, body, 0)
        plsc.subcore_barrier()

        def wb(k2, _):
            t = k2 * _NS + s

            @pl.when(t < nz)
            def _():
                r0 = t * zch
                pltpu.sync_copy(acc_sh.at[pl.ds(r0, zch)], stage_v)

                @pl.when(c == 0)
                def _():
                    pltpu.sync_copy(stage_v,
                                    dh_hbm.at[pl.ds(r0, zch), pl.ds(0, Hc)])

                @pl.when(c == 1)
                def _():
                    pltpu.sync_copy(stage_v,
                                    dh_hbm.at[pl.ds(r0, zch), pl.ds(Hc, Hc)])
            return 0

        lax.fori_loop(0, per_sz, wb, 0)

    return k_dmod, k_dh


def _conv_fwd_call(h, mod, src, dst):
    N, H = h.shape
    Hc = H // _NC
    E = src.shape[0]
    return _sc_conv_fwd_fn(N, H, E)(h[:, :Hc], h[:, Hc:], mod, src, dst)


def _conv_bwd_call(g, h, mod, src, dst):
    N, H = h.shape
    Hc = H // _NC
    E = src.shape[0]
    k_dmod, k_dh = _sc_conv_bwd_fn(N, H, E)
    g0, g1 = g[:, :Hc], g[:, Hc:]
    d_mod = k_dmod(g0, g1, h[:, :Hc], h[:, Hc:], src, dst)
    d_h = k_dh(g0, g1, mod, src, dst)
    return d_mod, d_h


def _make_conv_op():
    @jax.custom_vjp
    def conv_agg(h, mod, src, dst):
        return _conv_fwd_call(h, mod, src, dst)

    def cv_fwd(h, mod, src, dst):
        return _conv_fwd_call(h, mod, src, dst), (h, mod, src, dst)

    def cv_bwd(res, g):
        h, mod, src, dst = res
        d_mod, d_h = _conv_bwd_call(g, h, mod, src, dst)
        return d_h, d_mod, _f0(src), _f0(dst)

    conv_agg.defvjp(cv_fwd, cv_bwd)
    return conv_agg


def _gather_call(table, idx):
    N, H = table.shape
    (E,) = idx.shape
    return _sc_gather_fn(N, H, E)(table, idx)


def _scatter_call(msg, idx, N):
    E, H = msg.shape
    return _sc_scatter_fn(N, H, E)(msg, idx)


def _f0(idx):
    return np.zeros(idx.shape, jax.dtypes.float0)


def _make_sc_ops(N):
    @jax.custom_vjp
    def gather_rows(table, idx):
        return _gather_call(table, idx)

    def gather_fwd(table, idx):
        return _gather_call(table, idx), idx

    def gather_bwd(idx, g):
        return _scatter_call(g, idx, N), _f0(idx)

    gather_rows.defvjp(gather_fwd, gather_bwd)

    @jax.custom_vjp
    def seg_sum(msg, idx):
        return _scatter_call(msg, idx, N)

    def seg_fwd(msg, idx):
        return _scatter_call(msg, idx, N), idx

    def seg_bwd(idx, g):
        return _gather_call(g, idx), _f0(idx)

    seg_sum.defvjp(seg_fwd, seg_bwd)
    return gather_rows, seg_sum


def _mm(a, b):
    return jnp.dot(a, b, preferred_element_type=jnp.float32)


@functools.lru_cache(maxsize=None)
def _edge_mod_fwd_fn(E, H, K1, K2):
    def body(rbf_ref, shp_ref, wr_ref, wsp_ref, mod_ref):
        radial = _mm(rbf_ref[...], wr_ref[...])
        shw = _mm(shp_ref[...], wsp_ref[...])
        mod_ref[...] = radial * shw

    return pl.pallas_call(
        body,
        grid=(E // _BE,),
        in_specs=[
            pl.BlockSpec((_BE, K1), lambda i: (i, 0)),
            pl.BlockSpec((_BE, K2), lambda i: (i, 0)),
            pl.BlockSpec((K1, H), lambda i: (0, 0)),
            pl.BlockSpec((K2, H), lambda i: (0, 0)),
        ],
        out_specs=pl.BlockSpec((_BE, H), lambda i: (i, 0)),
        out_shape=jax.ShapeDtypeStruct((E, H), jnp.float32),
    )


@functools.lru_cache(maxsize=None)
def _edge_mod_bwd_fn(E, H, K1, K2):
    def body(g_ref, rbf_ref, shp_ref, wr_ref, wsp_ref, wrt_ref, wspt_ref,
             drbf_ref, dshp_ref):
        g = g_ref[...]
        radial = _mm(rbf_ref[...], wr_ref[...])
        shw = _mm(shp_ref[...], wsp_ref[...])
        drbf_ref[...] = _mm(g * shw, wrt_ref[...])
        dshp_ref[...] = _mm(g * radial, wspt_ref[...])

    return pl.pallas_call(
        body,
        grid=(E // _BE,),
        in_specs=[
            pl.BlockSpec((_BE, H), lambda i: (i, 0)),
            pl.BlockSpec((_BE, K1), lambda i: (i, 0)),
            pl.BlockSpec((_BE, K2), lambda i: (i, 0)),
            pl.BlockSpec((K1, H), lambda i: (0, 0)),
            pl.BlockSpec((K2, H), lambda i: (0, 0)),
            pl.BlockSpec((H, K1), lambda i: (0, 0)),
            pl.BlockSpec((H, K2), lambda i: (0, 0)),
        ],
        out_specs=[
            pl.BlockSpec((_BE, K1), lambda i: (i, 0)),
            pl.BlockSpec((_BE, K2), lambda i: (i, 0)),
        ],
        out_shape=[
            jax.ShapeDtypeStruct((E, K1), jnp.float32),
            jax.ShapeDtypeStruct((E, K2), jnp.float32),
        ],
    )


def _edge_mod_fwd_call(rbf, shp, wr, wsp):
    E, K1 = rbf.shape
    K2 = shp.shape[1]
    H = wr.shape[1]
    return _edge_mod_fwd_fn(E, H, K1, K2)(rbf, shp, wr, wsp)


def _edge_mod_bwd_call(g, rbf, shp, wr, wsp):
    E, K1 = rbf.shape
    K2 = shp.shape[1]
    H = wr.shape[1]
    return _edge_mod_bwd_fn(E, H, K1, K2)(g, rbf, shp, wr, wsp, wr.T, wsp.T)


@jax.custom_vjp
def _edge_mod(rbf, shp, wr, wsp):
    return _edge_mod_fwd_call(rbf, shp, wr, wsp)


def _edge_mod_f(rbf, shp, wr, wsp):
    return _edge_mod_fwd_call(rbf, shp, wr, wsp), (rbf, shp, wr, wsp)


def _edge_mod_b(res, g):
    rbf, shp, wr, wsp = res
    drbf, dshp = _edge_mod_bwd_call(g, rbf, shp, wr, wsp)
    return drbf, dshp, jnp.zeros_like(wr), jnp.zeros_like(wsp)


_edge_mod.defvjp(_edge_mod_f, _edge_mod_b)


@functools.lru_cache(maxsize=None)
def _node_upd_fwd_fn(N, H, A):
    def body(h_ref, agg_ref, xa_ref, ws_ref, wu_ref, wa_ref, pre_ref, hn_ref):
        pre = (_mm(h_ref[...], ws_ref[...]) + _mm(agg_ref[...], wu_ref[...])
               + _mm(xa_ref[...], wa_ref[...]))
        pre_ref[...] = pre
        hn_ref[...] = pre * jax.nn.sigmoid(pre)

    return pl.pallas_call(
        body,
        grid=(N // _BN,),
        in_specs=[
            pl.BlockSpec((_BN, H), lambda i: (i, 0)),
            pl.BlockSpec((_BN, H), lambda i: (i, 0)),
            pl.BlockSpec((_BN, A), lambda i: (i, 0)),
            pl.BlockSpec((H, H), lambda i: (0, 0)),
            pl.BlockSpec((H, H), lambda i: (0, 0)),
            pl.BlockSpec((A, H), lambda i: (0, 0)),
        ],
        out_specs=[
            pl.BlockSpec((_BN, H), lambda i: (i, 0)),
            pl.BlockSpec((_BN, H), lambda i: (i, 0)),
        ],
        out_shape=[
            jax.ShapeDtypeStruct((N, H), jnp.float32),
            jax.ShapeDtypeStruct((N, H), jnp.float32),
        ],
    )


@functools.lru_cache(maxsize=None)
def _node_upd_bwd_fn(N, H):
    def body(g_ref, pre_ref, wst_ref, wut_ref, dh_ref, dagg_ref):
        gp = g_ref[...] * _dsilu(pre_ref[...])
        dh_ref[...] = _mm(gp, wst_ref[...])
        dagg_ref[...] = _mm(gp, wut_ref[...])

    return pl.pallas_call(
        body,
        grid=(N // _BN,),
        in_specs=[
            pl.BlockSpec((_BN, H), lambda i: (i, 0)),
            pl.BlockSpec((_BN, H), lambda i: (i, 0)),
            pl.BlockSpec((H, H), lambda i: (0, 0)),
            pl.BlockSpec((H, H), lambda i: (0, 0)),
        ],
        out_specs=[
            pl.BlockSpec((_BN, H), lambda i: (i, 0)),
            pl.BlockSpec((_BN, H), lambda i: (i, 0)),
        ],
        out_shape=[
            jax.ShapeDtypeStruct((N, H), jnp.float32),
            jax.ShapeDtypeStruct((N, H), jnp.float32),
        ],
    )


def _node_upd_fwd_call(h, agg, xa, ws, wu, wa):
    N, H = h.shape
    A = xa.shape[1]
    return _node_upd_fwd_fn(N, H, A)(h, agg, xa, ws, wu, wa)


def _node_upd_bwd_call(g, pre, ws, wu):
    N, H = g.shape
    return _node_upd_bwd_fn(N, H)(g, pre, ws.T, wu.T)


@jax.custom_vjp
def _node_update(h, agg, xa, ws, wu, wa):
    _, hn = _node_upd_fwd_call(h, agg, xa, ws, wu, wa)
    return hn


def _node_update_f(h, agg, xa, ws, wu, wa):
    pre, hn = _node_upd_fwd_call(h, agg, xa, ws, wu, wa)
    return hn, (pre, ws, wu, xa.shape[1])


def _node_update_b(res, g):
    pre, ws, wu, A = res
    dh, dagg = _node_upd_bwd_call(g, pre, ws, wu)
    return (dh, dagg, jnp.zeros((g.shape[0], A), jnp.float32),
            jnp.zeros_like(ws), jnp.zeros_like(wu),
            jnp.zeros((A, g.shape[1]), jnp.float32))


_node_update.defvjp(_node_update_f, _node_update_b)


@functools.lru_cache(maxsize=None)
def _head_fwd_fn(N, H, P):
    def body(h_ref, w_ref, b_ref, z_ref, hp_ref):
        z = _mm(h_ref[...], w_ref[...]) + b_ref[...]
        z_ref[...] = z
        hp_ref[...] = z * jax.nn.sigmoid(z)

    return pl.pallas_call(
        body,
        grid=(N // _BN,),
        in_specs=[
            pl.BlockSpec((_BN, H), lambda i: (i, 0)),
            pl.BlockSpec((H, P), lambda i: (0, 0)),
            pl.BlockSpec((1, P), lambda i: (0, 0)),
        ],
        out_specs=[
            pl.BlockSpec((_BN, P), lambda i: (i, 0)),
            pl.BlockSpec((_BN, P), lambda i: (i, 0)),
        ],
        out_shape=[
            jax.ShapeDtypeStruct((N, P), jnp.float32),
            jax.ShapeDtypeStruct((N, P), jnp.float32),
        ],
    )


@functools.lru_cache(maxsize=None)
def _head_bwd_fn(N, H, P):
    def body(g_ref, z_ref, wt_ref, dh_ref):
        dh_ref[...] = _mm(g_ref[...] * _dsilu(z_ref[...]), wt_ref[...])

    return pl.pallas_call(
        body,
        grid=(N // _BN,),
        in_specs=[
            pl.BlockSpec((_BN, P), lambda i: (i, 0)),
            pl.BlockSpec((_BN, P), lambda i: (i, 0)),
            pl.BlockSpec((P, H), lambda i: (0, 0)),
        ],
        out_specs=pl.BlockSpec((_BN, H), lambda i: (i, 0)),
        out_shape=jax.ShapeDtypeStruct((N, H), jnp.float32),
    )


def _head_fwd_call(h, w, b):
    N, H = h.shape
    P = w.shape[1]
    return _head_fwd_fn(N, H, P)(h, w, b.reshape(1, P))


def _head_bwd_call(g, z, w):
    N, P = g.shape
    H = w.shape[0]
    return _head_bwd_fn(N, H, P)(g, z, w.T)


@jax.custom_vjp
def _head(h, w, b):
    _, hp = _head_fwd_call(h, w, b)
    return hp


def _head_f(h, w, b):
    z, hp = _head_fwd_call(h, w, b)
    return hp, (z, w)


def _head_b(res, g):
    z, w = res
    return _head_bwd_call(g, z, w), jnp.zeros_like(w), jnp.zeros((w.shape[1],), jnp.float32)


_head.defvjp(_head_f, _head_b)


@functools.lru_cache(maxsize=None)
def _embed_fn(N, A, H):
    def body(xa_ref, w_ref, b_ref, out_ref):
        out_ref[...] = _mm(xa_ref[...], w_ref[...]) + b_ref[...]

    return pl.pallas_call(
        body,
        grid=(N // _BN,),
        in_specs=[
            pl.BlockSpec((_BN, A), lambda i: (i, 0)),
            pl.BlockSpec((A, H), lambda i: (0, 0)),
            pl.BlockSpec((1, H), lambda i: (0, 0)),
        ],
        out_specs=pl.BlockSpec((_BN, H), lambda i: (i, 0)),
        out_shape=jax.ShapeDtypeStruct((N, H), jnp.float32),
    )


def _embed_call(xa, w, b):
    N, A = xa.shape
    H = w.shape[1]
    return _embed_fn(N, A, H)(xa, w, b.reshape(1, H))


def kernel(x, pos, edge_index, period_vec, batch, per_config_dataset_idx,
           elem_table, W_embed, b_embed, W_rbf, W_sh, W_self, W_upd, W_attr,
           W_p1, b_p1, W_p2, b_p2, scale, shift):
    N = pos.shape[0]
    E = edge_index.shape[1]
    H = W_embed.shape[1]
    G = per_config_dataset_idx.shape[0]
    nlayers, nrbf, _ = W_rbf.shape
    sh_dim = W_sh.shape[1]
    cutoff = 6.0

    src = edge_index[0]
    dst = edge_index[1]
    x_attr = elem_table[x]
    h0 = _embed_call(x_attr, W_embed, b_embed)
    k2 = 16
    W_shp = jnp.concatenate(
        [W_sh, jnp.zeros((nlayers, k2 - sh_dim, H), jnp.float32)], axis=1)

    conv_agg = _make_conv_op()
    edge_diff = _make_edge_diff(N, E)

    centers = jnp.linspace(0.0, cutoff, nrbf)
    width = cutoff / nrbf
    pool = (batch[:, None] == jnp.arange(G)[None, :]).astype(jnp.float32)

    def efn(pos_in):
        edge_vec = edge_diff(pos_in, period_vec, src, dst)
        lengths = jnp.sqrt(jnp.sum(edge_vec * edge_vec, axis=-1) + 1e-12)
        unit = edge_vec / lengths[:, None]
        ex, ey, ez = unit[:, 0], unit[:, 1], unit[:, 2]
        zero = jnp.zeros_like(ex)
        shp = jnp.stack(
            [jnp.ones_like(ex), ex, ey, ez, ex * ey, ey * ez, ez * ex,
             ex * ex - ey * ey, 3.0 * ez * ez - 1.0,
             zero, zero, zero, zero, zero, zero, zero], axis=-1)
        rbf = jnp.exp(-jnp.square((lengths[:, None] - centers[None, :]) / width))
        env = 0.5 * (jnp.cos(jnp.pi * jnp.clip(lengths / cutoff, 0.0, 1.0)) + 1.0)
        rbf = rbf * env[:, None]

        h = h0
        for l in range(nlayers):
            mod = _edge_mod(rbf, shp, W_rbf[l], W_shp[l])
            agg = conv_agg(h, mod, src, dst)
            h = _node_update(h, agg, x_attr, W_self[l], W_upd[l], W_attr[l])

        hp1 = _head(h, W_p1, b_p1)
        hp2 = _mm(hp1, W_p2) + b_p2
        graph_e = _mm(pool.T, hp2)
        energies_all = graph_e * scale + shift
        return energies_all[jnp.arange(G), per_config_dataset_idx]

    energies, vjp_fn = jax.vjp(efn, pos)
    forces = -vjp_fn(jnp.ones_like(energies))[0]
    return (energies, forces)


# speedup vs baseline: 1.4961x; 1.0648x over previous
import functools

import numpy as np
import jax
import jax.numpy as jnp
from jax import lax
from jax.experimental import pallas as pl
from jax.experimental.pallas import tpu as pltpu
from jax.experimental.pallas import tpu_sc as plsc

_NC = 2
_NS = 16
_NW = _NC * _NS

_BE = 2000
_BN = 1000
_CH = 128


def _dsilu(x):
    s = jax.nn.sigmoid(x)
    return s * (1.0 + x * (1.0 - s))


@functools.lru_cache(maxsize=None)
def _sc_gather_fn(N, H, E):
    nch = E // _CH
    assert nch * _CH == E
    per_w = -(-nch // _NW)
    mesh = plsc.VectorSubcoreMesh(core_axis_name="c", subcore_axis_name="s",
                                  num_cores=_NC, num_subcores=_NS)

    @functools.partial(
        pl.kernel, mesh=mesh,
        out_type=jax.ShapeDtypeStruct((E, H), jnp.float32),
        scratch_types=[
            pltpu.VMEM((_CH,), jnp.int32),
            pltpu.VMEM((_CH, H), jnp.float32),
            pltpu.SemaphoreType.DMA,
        ],
    )
    def k(table_hbm, idx_hbm, out_hbm, idx_v, rows_v, sem):
        wid = lax.axis_index("s") * _NC + lax.axis_index("c")

        def body(i, _):
            j = i * _NW + wid

            @pl.when(j < nch)
            def _():
                base = j * _CH
                pltpu.sync_copy(idx_hbm.at[pl.ds(base, _CH)], idx_v)
                pltpu.async_copy(table_hbm.at[idx_v], rows_v, sem).wait()
                pltpu.sync_copy(rows_v, out_hbm.at[pl.ds(base, _CH)])
            return 0

        lax.fori_loop(0, per_w, body, 0)

    return k


@functools.lru_cache(maxsize=None)
def _sc_scatter_fn(N, H, E):
    Hc = H // _NC
    nch = E // _CH
    assert nch * _CH == E
    per_s = -(-nch // _NS)
    zch = 80
    nz = N // zch
    assert nz * zch == N
    per_sz = -(-nz // _NS)
    mesh = plsc.VectorSubcoreMesh(core_axis_name="c", subcore_axis_name="s",
                                  num_cores=_NC, num_subcores=_NS)

    @functools.partial(
        pl.kernel, mesh=mesh,
        out_type=jax.ShapeDtypeStruct((N, H), jnp.float32),
        scratch_types=[
            pltpu.VMEM((_CH,), jnp.int32),
            pltpu.VMEM((_CH, Hc), jnp.float32),
            pltpu.VMEM((zch, Hc), jnp.float32),
            pltpu.VMEM_SHARED((N, Hc), jnp.float32),
            pltpu.SemaphoreType.DMA,
        ],
    )
    def k(msg_hbm, idx_hbm, out_hbm, idx_v, rows_v, stage_v, acc_sh, sem):
        c = lax.axis_index("c")
        s = lax.axis_index("s")

        def zrow(i, _):
            def zcol(j, _):
                stage_v[i, pl.ds(j * 16, 16)] = jnp.zeros((16,), jnp.float32)
                return 0
            lax.fori_loop(0, Hc // 16, zcol, 0)
            return 0
        lax.fori_loop(0, zch, zrow, 0)

        def zdma(k, _):
            t = k * _NS + s

            @pl.when(t < nz)
            def _():
                pltpu.sync_copy(stage_v, acc_sh.at[pl.ds(t * zch, zch)])
            return 0
        lax.fori_loop(0, per_sz, zdma, 0)
        plsc.subcore_barrier()

        def body(i, _):
            j = i * _NS + s

            @pl.when(j < nch)
            def _():
                base = j * _CH
                pltpu.sync_copy(idx_hbm.at[pl.ds(base, _CH)], idx_v)

                @pl.when(c == 0)
                def _():
                    pltpu.sync_copy(
                        msg_hbm.at[pl.ds(base, _CH), pl.ds(0, Hc)], rows_v)

                @pl.when(c == 1)
                def _():
                    pltpu.sync_copy(
                        msg_hbm.at[pl.ds(base, _CH), pl.ds(Hc, Hc)], rows_v)

                pltpu.sync_copy(rows_v, acc_sh.at[idx_v], add=True)
            return 0

        lax.fori_loop(0, per_s, body, 0)
        plsc.subcore_barrier()

        def wb(k, _):
            t = k * _NS + s

            @pl.when(t < nz)
            def _():
                r0 = t * zch
                pltpu.sync_copy(acc_sh.at[pl.ds(r0, zch)], stage_v)

                @pl.when(c == 0)
                def _():
                    pltpu.sync_copy(stage_v,
                                    out_hbm.at[pl.ds(r0, zch), pl.ds(0, Hc)])

                @pl.when(c == 1)
                def _():
                    pltpu.sync_copy(stage_v,
                                    out_hbm.at[pl.ds(r0, zch), pl.ds(Hc, Hc)])
            return 0

        lax.fori_loop(0, per_sz, wb, 0)

    return k


@functools.lru_cache(maxsize=None)
def _sc_scatter_pad_fn(N, E2, W):
    nch = E2 // _CH
    assert nch * _CH == E2
    nch_c = -(-nch // _NC)
    per_s = -(-nch_c // _NS)
    zch = 80
    nz = N // zch
    assert nz * zch == N
    per_sz = -(-nz // _NS)
    mesh = plsc.VectorSubcoreMesh(core_axis_name="c", subcore_axis_name="s",
                                  num_cores=_NC, num_subcores=_NS)

    @functools.partial(
        pl.kernel, mesh=mesh,
        out_type=jax.ShapeDtypeStruct((2, N, W), jnp.float32),
        scratch_types=[
            pltpu.VMEM((_CH,), jnp.int32),
            pltpu.VMEM((_CH, W), jnp.float32),
            pltpu.VMEM((zch, W), jnp.float32),
            pltpu.VMEM_SHARED((N, W), jnp.float32),
            pltpu.SemaphoreType.DMA,
        ],
    )
    def k(val_hbm, idx_hbm, out_hbm, idx_v, rows_v, stage_v, acc_sh, sem):
        c = lax.axis_index("c")
        s = lax.axis_index("s")

        def zrow(i, _):
            def zcol(j, _):
                stage_v[i, pl.ds(j * 16, 16)] = jnp.zeros((16,), jnp.float32)
                return 0
            lax.fori_loop(0, W // 16, zcol, 0)
            return 0
        lax.fori_loop(0, zch, zrow, 0)

        def zdma(k2, _):
            t = k2 * _NS + s

            @pl.when(t < nz)
            def _():
                pltpu.sync_copy(stage_v, acc_sh.at[pl.ds(t * zch, zch)])
            return 0
        lax.fori_loop(0, per_sz, zdma, 0)
        plsc.subcore_barrier()

        def body(i, _):
            kk = i * _NS + s
            j = kk * _NC + c

            @pl.when(j < nch)
            def _():
                base = j * _CH
                pltpu.sync_copy(idx_hbm.at[pl.ds(base, _CH)], idx_v)
                pltpu.sync_copy(val_hbm.at[pl.ds(base, _CH)], rows_v)
                pltpu.sync_copy(rows_v, acc_sh.at[idx_v], add=True)
            return 0

        lax.fori_loop(0, per_s, body, 0)
        plsc.subcore_barrier()

        def wb(k2, _):
            t = k2 * _NS + s

            @pl.when(t < nz)
            def _():
                r0 = t * zch
                pltpu.sync_copy(acc_sh.at[pl.ds(r0, zch)], stage_v)

                @pl.when(c == 0)
                def _():
                    pltpu.sync_copy(stage_v, out_hbm.at[0, pl.ds(r0, zch)])

                @pl.when(c == 1)
                def _():
                    pltpu.sync_copy(stage_v, out_hbm.at[1, pl.ds(r0, zch)])
            return 0

        lax.fori_loop(0, per_sz, wb, 0)

    return k


def _scatter_pad_call(val, idx, N):
    E2, W = val.shape
    parts = _sc_scatter_pad_fn(N, E2, W)(val, idx)
    return parts[0] + parts[1]


def _make_edge_diff(N, E):
    @jax.custom_vjp
    def edge_diff(pos, period_vec, src, dst):
        return pos[dst] - pos[src] + period_vec

    def ed_fwd(pos, period_vec, src, dst):
        return pos[dst] - pos[src] + period_vec, (src, dst)

    def ed_bwd(res, g):
        src, dst = res
        gp = jnp.pad(g, ((0, 0), (0, 125)))
        val = jnp.concatenate([gp, -gp], axis=0)
        idx = jnp.concatenate([dst, src], axis=0)
        d_pos = _scatter_pad_call(val, idx, N)[:, :3]
        return d_pos, g, _f0(src), _f0(dst)

    edge_diff.defvjp(ed_fwd, ed_bwd)
    return edge_diff


def _vmul_rows(dst_ref, a_ref, b_ref, rows, cols):
    def row(e, _):
        for jj in range(cols // 16):
            sl = pl.ds(jj * 16, 16)
            dst_ref[e, sl] = a_ref[e, sl] * b_ref[e, sl]
        return 0
    lax.fori_loop(0, rows, row, 0)


@functools.lru_cache(maxsize=None)
def _sc_conv_fwd_fn(N, H, E):
    Hc = H // _NC
    CHF = 64
    nch = E // CHF
    assert nch * CHF == E
    per_s = -(-nch // _NS)
    zch = 40
    nz = N // zch
    assert nz * zch == N
    per_sz = -(-nz // _NS)
    mesh = plsc.VectorSubcoreMesh(core_axis_name="c", subcore_axis_name="s",
                                  num_cores=_NC, num_subcores=_NS)

    @functools.partial(
        pl.kernel, mesh=mesh,
        out_type=jax.ShapeDtypeStruct((N, H), jnp.float32),
        scratch_types=[
            pltpu.VMEM((CHF,), jnp.int32),
            pltpu.VMEM((CHF,), jnp.int32),
            pltpu.VMEM((CHF,), jnp.int32),
            pltpu.VMEM((CHF,), jnp.int32),
            pltpu.VMEM((CHF, Hc), jnp.float32),
            pltpu.VMEM((CHF, Hc), jnp.float32),
            pltpu.VMEM((CHF, Hc), jnp.float32),
            pltpu.VMEM((CHF, Hc), jnp.float32),
            pltpu.VMEM((zch, Hc), jnp.float32),
            pltpu.VMEM_SHARED((N, Hc), jnp.float32),
            pltpu.SemaphoreType.DMA,
            pltpu.SemaphoreType.DMA,
            pltpu.SemaphoreType.DMA,
            pltpu.SemaphoreType.DMA,
        ],
    )
    def k(h0_hbm, h1_hbm, mod_hbm, src_hbm, dst_hbm, out_hbm,
          si0, di0, si1, di1, g0, m0, g1, m1, stage_v, acc_sh,
          sg0, sm0, sg1, sm1):
        c = lax.axis_index("c")
        s = lax.axis_index("s")
        bufs = ((si0, di0, g0, m0, sg0, sm0), (si1, di1, g1, m1, sg1, sm1))

        def zrow(i, _):
            def zcol(j, _):
                stage_v[i, pl.ds(j * 16, 16)] = jnp.zeros((16,), jnp.float32)
                return 0
            lax.fori_loop(0, Hc // 16, zcol, 0)
            return 0
        lax.fori_loop(0, zch, zrow, 0)

        def zdma(k2, _):
            t = k2 * _NS + s

            @pl.when(t < nz)
            def _():
                pltpu.sync_copy(stage_v, acc_sh.at[pl.ds(t * zch, zch)])
            return 0
        lax.fori_loop(0, per_sz, zdma, 0)
        plsc.subcore_barrier()

        def prefetch(i, b):
            si, di, g, m, sg, sm = bufs[b]
            j = i * _NS + s

            @pl.when(j < nch)
            def _():
                base = j * CHF
                pltpu.sync_copy(src_hbm.at[pl.ds(base, CHF)], si)
                pltpu.sync_copy(dst_hbm.at[pl.ds(base, CHF)], di)

                @pl.when(c == 0)
                def _():
                    pltpu.async_copy(h0_hbm.at[si], g, sg)
                    pltpu.async_copy(
                        mod_hbm.at[pl.ds(base, CHF), pl.ds(0, Hc)], m, sm)

                @pl.when(c == 1)
                def _():
                    pltpu.async_copy(h1_hbm.at[si], g, sg)
                    pltpu.async_copy(
                        mod_hbm.at[pl.ds(base, CHF), pl.ds(Hc, Hc)], m, sm)

        def process(i, b):
            si, di, g, m, sg, sm = bufs[b]
            j = i * _NS + s

            @pl.when(j < nch)
            def _():
                base = j * CHF

                @pl.when(c == 0)
                def _():
                    pltpu.make_async_copy(h0_hbm.at[si], g, sg).wait()
                    pltpu.make_async_copy(
                        mod_hbm.at[pl.ds(base, CHF), pl.ds(0, Hc)], m, sm).wait()

                @pl.when(c == 1)
                def _():
                    pltpu.make_async_copy(h1_hbm.at[si], g, sg).wait()
                    pltpu.make_async_copy(
                        mod_hbm.at[pl.ds(base, CHF), pl.ds(Hc, Hc)], m, sm).wait()

                _vmul_rows(m, m, g, CHF, Hc)
                pltpu.sync_copy(m, acc_sh.at[di], add=True)

        prefetch(0, 0)

        def body(t, _):
            i0 = 2 * t
            prefetch(i0 + 1, 1)
            process(i0, 0)
            prefetch(i0 + 2, 0)
            process(i0 + 1, 1)
            return 0

        lax.fori_loop(0, (per_s + 1) // 2, body, 0)
        plsc.subcore_barrier()

        def wb(k2, _):
            t = k2 * _NS + s

            @pl.when(t < nz)
            def _():
                r0 = t * zch
                pltpu.sync_copy(acc_sh.at[pl.ds(r0, zch)], stage_v)

                @pl.when(c == 0)
                def _():
                    pltpu.sync_copy(stage_v,
                                    out_hbm.at[pl.ds(r0, zch), pl.ds(0, Hc)])

                @pl.when(c == 1)
                def _():
                    pltpu.sync_copy(stage_v,
                                    out_hbm.at[pl.ds(r0, zch), pl.ds(Hc, Hc)])
            return 0

        lax.fori_loop(0, per_sz, wb, 0)

    return k


@functools.lru_cache(maxsize=None)
def _sc_conv_bwd_fn(N, H, E):
    Hc = H // _NC
    nch = E // _CH
    assert nch * _CH == E
    per_s = -(-nch // _NS)
    zch = 80
    nz = N // zch
    assert nz * zch == N
    per_sz = -(-nz // _NS)
    mesh = plsc.VectorSubcoreMesh(core_axis_name="c", subcore_axis_name="s",
                                  num_cores=_NC, num_subcores=_NS)

    @functools.partial(
        pl.kernel, mesh=mesh,
        out_type=jax.ShapeDtypeStruct((E, H), jnp.float32),
        scratch_types=[
            pltpu.VMEM((_CH,), jnp.int32),
            pltpu.VMEM((_CH,), jnp.int32),
            pltpu.VMEM((_CH, Hc), jnp.float32),
            pltpu.VMEM((_CH, Hc), jnp.float32),
            pltpu.SemaphoreType.DMA,
            pltpu.SemaphoreType.DMA,
        ],
    )
    def k_dmod(da0_hbm, da1_hbm, h0_hbm, h1_hbm, src_hbm, dst_hbm,
               dmod_hbm, si_v, di_v, t_v, g_v, sem1, sem2):
        c = lax.axis_index("c")
        s = lax.axis_index("s")

        def body(i, _):
            j = i * _NS + s

            @pl.when(j < nch)
            def _():
                base = j * _CH
                pltpu.sync_copy(src_hbm.at[pl.ds(base, _CH)], si_v)
                pltpu.sync_copy(dst_hbm.at[pl.ds(base, _CH)], di_v)

                @pl.when(c == 0)
                def _():
                    cp1 = pltpu.async_copy(da0_hbm.at[di_v], t_v, sem1)
                    cp2 = pltpu.async_copy(h0_hbm.at[si_v], g_v, sem2)
                    cp1.wait()
                    cp2.wait()
                    _vmul_rows(g_v, g_v, t_v, _CH, Hc)
                    pltpu.sync_copy(
                        g_v, dmod_hbm.at[pl.ds(base, _CH), pl.ds(0, Hc)])

                @pl.when(c == 1)
                def _():
                    cp1 = pltpu.async_copy(da1_hbm.at[di_v], t_v, sem1)
                    cp2 = pltpu.async_copy(h1_hbm.at[si_v], g_v, sem2)
                    cp1.wait()
                    cp2.wait()
                    _vmul_rows(g_v, g_v, t_v, _CH, Hc)
                    pltpu.sync_copy(
                        g_v, dmod_hbm.at[pl.ds(base, _CH), pl.ds(Hc, Hc)])
            return 0

        lax.fori_loop(0, per_s, body, 0)

    @functools.partial(
        pl.kernel, mesh=mesh,
        out_type=jax.ShapeDtypeStruct((N, H), jnp.float32),
        scratch_types=[
            pltpu.VMEM((_CH,), jnp.int32),
            pltpu.VMEM((_CH,), jnp.int32),
            pltpu.VMEM((_CH, Hc), jnp.float32),
            pltpu.VMEM((_CH, Hc), jnp.float32),
            pltpu.VMEM((zch, Hc), jnp.float32),
            pltpu.VMEM_SHARED((N, Hc), jnp.float32),
            pltpu.SemaphoreType.DMA,
        ],
    )
    def k_dh(da0_hbm, da1_hbm, mod_hbm, src_hbm, dst_hbm, dh_hbm,
             si_v, di_v, t_v, m_v, stage_v, acc_sh, sem1):
        c = lax.axis_index("c")
        s = lax.axis_index("s")

        def zrow(i, _):
            def zcol(j, _):
                stage_v[i, pl.ds(j * 16, 16)] = jnp.zeros((16,), jnp.float32)
                return 0
            lax.fori_loop(0, Hc // 16, zcol, 0)
            return 0
        lax.fori_loop(0, zch, zrow, 0)

        def zdma(k2, _):
            t = k2 * _NS + s

            @pl.when(t < nz)
            def _():
                pltpu.sync_copy(stage_v, acc_sh.at[pl.ds(t * zch, zch)])
            return 0
        lax.fori_loop(0, per_sz, zdma, 0)
        plsc.subcore_barrier()

        def body(i, _):
            j = i * _NS + s

            @pl.when(j < nch)
            def _():
                base = j * _CH
                pltpu.sync_copy(src_hbm.at[pl.ds(base, _CH)], si_v)
                pltpu.sync_copy(dst_hbm.at[pl.ds(base, _CH)], di_v)

                @pl.when(c == 0)
                def _():
                    cp1 = pltpu.async_copy(da0_hbm.at[di_v], t_v, sem1)
                    pltpu.sync_copy(
                        mod_hbm.at[pl.ds(base, _CH), pl.ds(0, Hc)], m_v)
                    cp1.wait()

                @pl.when(c == 1)
                def _():
                    cp1 = pltpu.async_copy(da1_hbm.at[di_v], t_v, sem1)
                    pltpu.sync_copy(
                        mod_hbm.at[pl.ds(base, _CH), pl.ds(Hc, Hc)], m_v)
                    cp1.wait()

                _vmul_rows(m_v, m_v, t_v, _CH, Hc)
                pltpu.sync_copy(m_v, acc_sh.at[si_v], add=True)
            return 0

        lax.fori_loop(0, per_s, body, 0)
        plsc.subcore_barrier()

        def wb(k2, _):
            t = k2 * _NS + s

            @pl.when(t < nz)
            def _():
                r0 = t * zch
                pltpu.sync_copy(acc_sh.at[pl.ds(r0, zch)], stage_v)

                @pl.when(c == 0)
                def _():
                    pltpu.sync_copy(stage_v,
                                    dh_hbm.at[pl.ds(r0, zch), pl.ds(0, Hc)])

                @pl.when(c == 1)
                def _():
                    pltpu.sync_copy(stage_v,
                                    dh_hbm.at[pl.ds(r0, zch), pl.ds(Hc, Hc)])
            return 0

        lax.fori_loop(0, per_sz, wb, 0)

    return k_dmod, k_dh


def _conv_fwd_call(h, mod, src, dst):
    N, H = h.shape
    Hc = H // _NC
    E = src.shape[0]
    return _sc_conv_fwd_fn(N, H, E)(h[:, :Hc], h[:, Hc:], mod, src, dst)


def _conv_bwd_call(g, h, mod, src, dst):
    N, H = h.shape
    Hc = H // _NC
    E = src.shape[0]
    k_dmod, k_dh = _sc_conv_bwd_fn(N, H, E)
    g0, g1 = g[:, :Hc], g[:, Hc:]
    d_mod = k_dmod(g0, g1, h[:, :Hc], h[:, Hc:], src, dst)
    d_h = k_dh(g0, g1, mod, src, dst)
    return d_mod, d_h


def _dmod_call(g, h, src, dst):
    N, H = h.shape
    Hc = H // _NC
    E = src.shape[0]
    k_dmod, _ = _sc_conv_bwd_fn(N, H, E)
    return k_dmod(g[:, :Hc], g[:, Hc:], h[:, :Hc], h[:, Hc:], src, dst)


def _make_conv_op(need_dh):
    @jax.custom_vjp
    def conv_agg(h, mod, src, dst):
        return _conv_fwd_call(h, mod, src, dst)

    def cv_fwd(h, mod, src, dst):
        return _conv_fwd_call(h, mod, src, dst), (h, mod, src, dst)

    def cv_bwd(res, g):
        h, mod, src, dst = res
        if need_dh:
            d_mod, d_h = _conv_bwd_call(g, h, mod, src, dst)
        else:
            d_mod = _dmod_call(g, h, src, dst)
            d_h = jnp.zeros_like(h)
        return d_h, d_mod, _f0(src), _f0(dst)

    conv_agg.defvjp(cv_fwd, cv_bwd)
    return conv_agg


def _gather_call(table, idx):
    N, H = table.shape
    (E,) = idx.shape
    return _sc_gather_fn(N, H, E)(table, idx)


def _scatter_call(msg, idx, N):
    E, H = msg.shape
    return _sc_scatter_fn(N, H, E)(msg, idx)


def _f0(idx):
    return np.zeros(idx.shape, jax.dtypes.float0)


def _make_sc_ops(N):
    @jax.custom_vjp
    def gather_rows(table, idx):
        return _gather_call(table, idx)

    def gather_fwd(table, idx):
        return _gather_call(table, idx), idx

    def gather_bwd(idx, g):
        return _scatter_call(g, idx, N), _f0(idx)

    gather_rows.defvjp(gather_fwd, gather_bwd)

    @jax.custom_vjp
    def seg_sum(msg, idx):
        return _scatter_call(msg, idx, N)

    def seg_fwd(msg, idx):
        return _scatter_call(msg, idx, N), idx

    def seg_bwd(idx, g):
        return _gather_call(g, idx), _f0(idx)

    seg_sum.defvjp(seg_fwd, seg_bwd)
    return gather_rows, seg_sum


def _mm(a, b):
    return jnp.dot(a, b, preferred_element_type=jnp.float32)


@functools.lru_cache(maxsize=None)
def _edge_mod_fwd_fn(E, H, K1, K2):
    def body(rbf_ref, shp_ref, wr_ref, wsp_ref, mod_ref):
        radial = _mm(rbf_ref[...], wr_ref[...])
        shw = _mm(shp_ref[...], wsp_ref[...])
        mod_ref[...] = radial * shw

    return pl.pallas_call(
        body,
        grid=(E // _BE,),
        in_specs=[
            pl.BlockSpec((_BE, K1), lambda i: (i, 0)),
            pl.BlockSpec((_BE, K2), lambda i: (i, 0)),
            pl.BlockSpec((K1, H), lambda i: (0, 0)),
            pl.BlockSpec((K2, H), lambda i: (0, 0)),
        ],
        out_specs=pl.BlockSpec((_BE, H), lambda i: (i, 0)),
        out_shape=jax.ShapeDtypeStruct((E, H), jnp.float32),
    )


@functools.lru_cache(maxsize=None)
def _edge_mod_bwd_fn(E, H, K1, K2):
    def body(g_ref, rbf_ref, shp_ref, wr_ref, wsp_ref, wrt_ref, wspt_ref,
             drbf_ref, dshp_ref):
        g = g_ref[...]
        radial = _mm(rbf_ref[...], wr_ref[...])
        shw = _mm(shp_ref[...], wsp_ref[...])
        drbf_ref[...] = _mm(g * shw, wrt_ref[...])
        dshp_ref[...] = _mm(g * radial, wspt_ref[...])

    return pl.pallas_call(
        body,
        grid=(E // _BE,),
        in_specs=[
            pl.BlockSpec((_BE, H), lambda i: (i, 0)),
            pl.BlockSpec((_BE, K1), lambda i: (i, 0)),
            pl.BlockSpec((_BE, K2), lambda i: (i, 0)),
            pl.BlockSpec((K1, H), lambda i: (0, 0)),
            pl.BlockSpec((K2, H), lambda i: (0, 0)),
            pl.BlockSpec((H, K1), lambda i: (0, 0)),
            pl.BlockSpec((H, K2), lambda i: (0, 0)),
        ],
        out_specs=[
            pl.BlockSpec((_BE, K1), lambda i: (i, 0)),
            pl.BlockSpec((_BE, K2), lambda i: (i, 0)),
        ],
        out_shape=[
            jax.ShapeDtypeStruct((E, K1), jnp.float32),
            jax.ShapeDtypeStruct((E, K2), jnp.float32),
        ],
    )


def _edge_mod_fwd_call(rbf, shp, wr, wsp):
    E, K1 = rbf.shape
    K2 = shp.shape[1]
    H = wr.shape[1]
    return _edge_mod_fwd_fn(E, H, K1, K2)(rbf, shp, wr, wsp)


def _edge_mod_bwd_call(g, rbf, shp, wr, wsp):
    E, K1 = rbf.shape
    K2 = shp.shape[1]
    H = wr.shape[1]
    return _edge_mod_bwd_fn(E, H, K1, K2)(g, rbf, shp, wr, wsp, wr.T, wsp.T)


@jax.custom_vjp
def _edge_mod(rbf, shp, wr, wsp):
    return _edge_mod_fwd_call(rbf, shp, wr, wsp)


def _edge_mod_f(rbf, shp, wr, wsp):
    return _edge_mod_fwd_call(rbf, shp, wr, wsp), (rbf, shp, wr, wsp)


def _edge_mod_b(res, g):
    rbf, shp, wr, wsp = res
    drbf, dshp = _edge_mod_bwd_call(g, rbf, shp, wr, wsp)
    return drbf, dshp, jnp.zeros_like(wr), jnp.zeros_like(wsp)


_edge_mod.defvjp(_edge_mod_f, _edge_mod_b)


@functools.lru_cache(maxsize=None)
def _node_upd_fwd_fn(N, H, A):
    def body(h_ref, agg_ref, xa_ref, ws_ref, wu_ref, wa_ref, pre_ref, hn_ref):
        pre = (_mm(h_ref[...], ws_ref[...]) + _mm(agg_ref[...], wu_ref[...])
               + _mm(xa_ref[...], wa_ref[...]))
        pre_ref[...] = pre
        hn_ref[...] = pre * jax.nn.sigmoid(pre)

    return pl.pallas_call(
        body,
        grid=(N // _BN,),
        in_specs=[
            pl.BlockSpec((_BN, H), lambda i: (i, 0)),
            pl.BlockSpec((_BN, H), lambda i: (i, 0)),
            pl.BlockSpec((_BN, A), lambda i: (i, 0)),
            pl.BlockSpec((H, H), lambda i: (0, 0)),
            pl.BlockSpec((H, H), lambda i: (0, 0)),
            pl.BlockSpec((A, H), lambda i: (0, 0)),
        ],
        out_specs=[
            pl.BlockSpec((_BN, H), lambda i: (i, 0)),
            pl.BlockSpec((_BN, H), lambda i: (i, 0)),
        ],
        out_shape=[
            jax.ShapeDtypeStruct((N, H), jnp.float32),
            jax.ShapeDtypeStruct((N, H), jnp.float32),
        ],
    )


@functools.lru_cache(maxsize=None)
def _node_upd_bwd_fn(N, H):
    def body(g_ref, pre_ref, wst_ref, wut_ref, dh_ref, dagg_ref):
        gp = g_ref[...] * _dsilu(pre_ref[...])
        dh_ref[...] = _mm(gp, wst_ref[...])
        dagg_ref[...] = _mm(gp, wut_ref[...])

    return pl.pallas_call(
        body,
        grid=(N // _BN,),
        in_specs=[
            pl.BlockSpec((_BN, H), lambda i: (i, 0)),
            pl.BlockSpec((_BN, H), lambda i: (i, 0)),
            pl.BlockSpec((H, H), lambda i: (0, 0)),
            pl.BlockSpec((H, H), lambda i: (0, 0)),
        ],
        out_specs=[
            pl.BlockSpec((_BN, H), lambda i: (i, 0)),
            pl.BlockSpec((_BN, H), lambda i: (i, 0)),
        ],
        out_shape=[
            jax.ShapeDtypeStruct((N, H), jnp.float32),
            jax.ShapeDtypeStruct((N, H), jnp.float32),
        ],
    )


def _node_upd_fwd_call(h, agg, xa, ws, wu, wa):
    N, H = h.shape
    A = xa.shape[1]
    return _node_upd_fwd_fn(N, H, A)(h, agg, xa, ws, wu, wa)


def _node_upd_bwd_call(g, pre, ws, wu):
    N, H = g.shape
    return _node_upd_bwd_fn(N, H)(g, pre, ws.T, wu.T)


@jax.custom_vjp
def _node_update(h, agg, xa, ws, wu, wa):
    _, hn = _node_upd_fwd_call(h, agg, xa, ws, wu, wa)
    return hn


def _node_update_f(h, agg, xa, ws, wu, wa):
    pre, hn = _node_upd_fwd_call(h, agg, xa, ws, wu, wa)
    return hn, (pre, ws, wu, xa.shape[1])


def _node_update_b(res, g):
    pre, ws, wu, A = res
    dh, dagg = _node_upd_bwd_call(g, pre, ws, wu)
    return (dh, dagg, jnp.zeros((g.shape[0], A), jnp.float32),
            jnp.zeros_like(ws), jnp.zeros_like(wu),
            jnp.zeros((A, g.shape[1]), jnp.float32))


_node_update.defvjp(_node_update_f, _node_update_b)


@functools.lru_cache(maxsize=None)
def _head_fwd_fn(N, H, P):
    def body(h_ref, w_ref, b_ref, z_ref, hp_ref):
        z = _mm(h_ref[...], w_ref[...]) + b_ref[...]
        z_ref[...] = z
        hp_ref[...] = z * jax.nn.sigmoid(z)

    return pl.pallas_call(
        body,
        grid=(N // _BN,),
        in_specs=[
            pl.BlockSpec((_BN, H), lambda i: (i, 0)),
            pl.BlockSpec((H, P), lambda i: (0, 0)),
            pl.BlockSpec((1, P), lambda i: (0, 0)),
        ],
        out_specs=[
            pl.BlockSpec((_BN, P), lambda i: (i, 0)),
            pl.BlockSpec((_BN, P), lambda i: (i, 0)),
        ],
        out_shape=[
            jax.ShapeDtypeStruct((N, P), jnp.float32),
            jax.ShapeDtypeStruct((N, P), jnp.float32),
        ],
    )


@functools.lru_cache(maxsize=None)
def _head_bwd_fn(N, H, P):
    def body(g_ref, z_ref, wt_ref, dh_ref):
        dh_ref[...] = _mm(g_ref[...] * _dsilu(z_ref[...]), wt_ref[...])

    return pl.pallas_call(
        body,
        grid=(N // _BN,),
        in_specs=[
            pl.BlockSpec((_BN, P), lambda i: (i, 0)),
            pl.BlockSpec((_BN, P), lambda i: (i, 0)),
            pl.BlockSpec((P, H), lambda i: (0, 0)),
        ],
        out_specs=pl.BlockSpec((_BN, H), lambda i: (i, 0)),
        out_shape=jax.ShapeDtypeStruct((N, H), jnp.float32),
    )


def _head_fwd_call(h, w, b):
    N, H = h.shape
    P = w.shape[1]
    return _head_fwd_fn(N, H, P)(h, w, b.reshape(1, P))


def _head_bwd_call(g, z, w):
    N, P = g.shape
    H = w.shape[0]
    return _head_bwd_fn(N, H, P)(g, z, w.T)


@jax.custom_vjp
def _head(h, w, b):
    _, hp = _head_fwd_call(h, w, b)
    return hp


def _head_f(h, w, b):
    z, hp = _head_fwd_call(h, w, b)
    return hp, (z, w)


def _head_b(res, g):
    z, w = res
    return _head_bwd_call(g, z, w), jnp.zeros_like(w), jnp.zeros((w.shape[1],), jnp.float32)


_head.defvjp(_head_f, _head_b)


@functools.lru_cache(maxsize=None)
def _embed_fn(N, A, H):
    def body(xa_ref, w_ref, b_ref, out_ref):
        out_ref[...] = _mm(xa_ref[...], w_ref[...]) + b_ref[...]

    return pl.pallas_call(
        body,
        grid=(N // _BN,),
        in_specs=[
            pl.BlockSpec((_BN, A), lambda i: (i, 0)),
            pl.BlockSpec((A, H), lambda i: (0, 0)),
            pl.BlockSpec((1, H), lambda i: (0, 0)),
        ],
        out_specs=pl.BlockSpec((_BN, H), lambda i: (i, 0)),
        out_shape=jax.ShapeDtypeStruct((N, H), jnp.float32),
    )


def _embed_call(xa, w, b):
    N, A = xa.shape
    H = w.shape[1]
    return _embed_fn(N, A, H)(xa, w, b.reshape(1, H))


def kernel(x, pos, edge_index, period_vec, batch, per_config_dataset_idx,
           elem_table, W_embed, b_embed, W_rbf, W_sh, W_self, W_upd, W_attr,
           W_p1, b_p1, W_p2, b_p2, scale, shift):
    N = pos.shape[0]
    E = edge_index.shape[1]
    H = W_embed.shape[1]
    G = per_config_dataset_idx.shape[0]
    nlayers, nrbf, _ = W_rbf.shape
    sh_dim = W_sh.shape[1]
    cutoff = 6.0

    src = edge_index[0]
    dst = edge_index[1]
    x_attr = elem_table[x]
    h0 = _embed_call(x_attr, W_embed, b_embed)
    k2 = 16
    W_shp = jnp.concatenate(
        [W_sh, jnp.zeros((nlayers, k2 - sh_dim, H), jnp.float32)], axis=1)

    conv_agg = _make_conv_op(True)
    conv_agg0 = _make_conv_op(False)
    edge_diff = _make_edge_diff(N, E)

    centers = jnp.linspace(0.0, cutoff, nrbf)
    width = cutoff / nrbf
    pool = (batch[:, None] == jnp.arange(G)[None, :]).astype(jnp.float32)

    def efn(pos_in):
        edge_vec = edge_diff(pos_in, period_vec, src, dst)
        lengths = jnp.sqrt(jnp.sum(edge_vec * edge_vec, axis=-1) + 1e-12)
        unit = edge_vec / lengths[:, None]
        ex, ey, ez = unit[:, 0], unit[:, 1], unit[:, 2]
        zero = jnp.zeros_like(ex)
        shp = jnp.stack(
            [jnp.ones_like(ex), ex, ey, ez, ex * ey, ey * ez, ez * ex,
             ex * ex - ey * ey, 3.0 * ez * ez - 1.0,
             zero, zero, zero, zero, zero, zero, zero], axis=-1)
        rbf = jnp.exp(-jnp.square((lengths[:, None] - centers[None, :]) / width))
        env = 0.5 * (jnp.cos(jnp.pi * jnp.clip(lengths / cutoff, 0.0, 1.0)) + 1.0)
        rbf = rbf * env[:, None]

        h = h0
        for l in range(nlayers):
            mod = _edge_mod(rbf, shp, W_rbf[l], W_shp[l])
            agg = (conv_agg if l > 0 else conv_agg0)(h, mod, src, dst)
            h = _node_update(h, agg, x_attr, W_self[l], W_upd[l], W_attr[l])

        hp1 = _head(h, W_p1, b_p1)
        hp2 = _mm(hp1, W_p2) + b_p2
        graph_e = _mm(pool.T, hp2)
        energies_all = graph_e * scale + shift
        return energies_all[jnp.arange(G), per_config_dataset_idx]

    energies, vjp_fn = jax.vjp(efn, pos)
    forces = -vjp_fn(jnp.ones_like(energies))[0]
    return (energies, forces)


# speedup vs baseline: 1.6346x; 1.0925x over previous
import functools

import numpy as np
import jax
import jax.numpy as jnp
from jax import lax
from jax.experimental import pallas as pl
from jax.experimental.pallas import tpu as pltpu
from jax.experimental.pallas import tpu_sc as plsc

_NC = 2
_NS = 16
_NW = _NC * _NS

_BE = 2000
_BN = 1000
_CH = 128


def _dsilu(x):
    s = jax.nn.sigmoid(x)
    return s * (1.0 + x * (1.0 - s))


@functools.lru_cache(maxsize=None)
def _sc_gather_fn(N, H, E):
    nch = E // _CH
    assert nch * _CH == E
    per_w = -(-nch // _NW)
    mesh = plsc.VectorSubcoreMesh(core_axis_name="c", subcore_axis_name="s",
                                  num_cores=_NC, num_subcores=_NS)

    @functools.partial(
        pl.kernel, mesh=mesh,
        out_type=jax.ShapeDtypeStruct((E, H), jnp.float32),
        scratch_types=[
            pltpu.VMEM((_CH,), jnp.int32),
            pltpu.VMEM((_CH, H), jnp.float32),
            pltpu.SemaphoreType.DMA,
        ],
    )
    def k(table_hbm, idx_hbm, out_hbm, idx_v, rows_v, sem):
        wid = lax.axis_index("s") * _NC + lax.axis_index("c")

        def body(i, _):
            j = i * _NW + wid

            @pl.when(j < nch)
            def _():
                base = j * _CH
                pltpu.sync_copy(idx_hbm.at[pl.ds(base, _CH)], idx_v)
                pltpu.async_copy(table_hbm.at[idx_v], rows_v, sem).wait()
                pltpu.sync_copy(rows_v, out_hbm.at[pl.ds(base, _CH)])
            return 0

        lax.fori_loop(0, per_w, body, 0)

    return k


@functools.lru_cache(maxsize=None)
def _sc_scatter_fn(N, H, E):
    Hc = H // _NC
    nch = E // _CH
    assert nch * _CH == E
    per_s = -(-nch // _NS)
    zch = 80
    nz = N // zch
    assert nz * zch == N
    per_sz = -(-nz // _NS)
    mesh = plsc.VectorSubcoreMesh(core_axis_name="c", subcore_axis_name="s",
                                  num_cores=_NC, num_subcores=_NS)

    @functools.partial(
        pl.kernel, mesh=mesh,
        out_type=jax.ShapeDtypeStruct((N, H), jnp.float32),
        scratch_types=[
            pltpu.VMEM((_CH,), jnp.int32),
            pltpu.VMEM((_CH, Hc), jnp.float32),
            pltpu.VMEM((zch, Hc), jnp.float32),
            pltpu.VMEM_SHARED((N, Hc), jnp.float32),
            pltpu.SemaphoreType.DMA,
        ],
    )
    def k(msg_hbm, idx_hbm, out_hbm, idx_v, rows_v, stage_v, acc_sh, sem):
        c = lax.axis_index("c")
        s = lax.axis_index("s")

        def zrow(i, _):
            def zcol(j, _):
                stage_v[i, pl.ds(j * 16, 16)] = jnp.zeros((16,), jnp.float32)
                return 0
            lax.fori_loop(0, Hc // 16, zcol, 0)
            return 0
        lax.fori_loop(0, zch, zrow, 0)

        def zdma(k, _):
            t = k * _NS + s

            @pl.when(t < nz)
            def _():
                pltpu.sync_copy(stage_v, acc_sh.at[pl.ds(t * zch, zch)])
            return 0
        lax.fori_loop(0, per_sz, zdma, 0)
        plsc.subcore_barrier()

        def body(i, _):
            j = i * _NS + s

            @pl.when(j < nch)
            def _():
                base = j * _CH
                pltpu.sync_copy(idx_hbm.at[pl.ds(base, _CH)], idx_v)

                @pl.when(c == 0)
                def _():
                    pltpu.sync_copy(
                        msg_hbm.at[pl.ds(base, _CH), pl.ds(0, Hc)], rows_v)

                @pl.when(c == 1)
                def _():
                    pltpu.sync_copy(
                        msg_hbm.at[pl.ds(base, _CH), pl.ds(Hc, Hc)], rows_v)

                pltpu.sync_copy(rows_v, acc_sh.at[idx_v], add=True)
            return 0

        lax.fori_loop(0, per_s, body, 0)
        plsc.subcore_barrier()

        def wb(k, _):
            t = k * _NS + s

            @pl.when(t < nz)
            def _():
                r0 = t * zch
                pltpu.sync_copy(acc_sh.at[pl.ds(r0, zch)], stage_v)

                @pl.when(c == 0)
                def _():
                    pltpu.sync_copy(stage_v,
                                    out_hbm.at[pl.ds(r0, zch), pl.ds(0, Hc)])

                @pl.when(c == 1)
                def _():
                    pltpu.sync_copy(stage_v,
                                    out_hbm.at[pl.ds(r0, zch), pl.ds(Hc, Hc)])
            return 0

        lax.fori_loop(0, per_sz, wb, 0)

    return k


@functools.lru_cache(maxsize=None)
def _sc_scatter_pad_fn(N, E2, W):
    nch = E2 // _CH
    assert nch * _CH == E2
    nch_c = -(-nch // _NC)
    per_s = -(-nch_c // _NS)
    zch = 80
    nz = N // zch
    assert nz * zch == N
    per_sz = -(-nz // _NS)
    mesh = plsc.VectorSubcoreMesh(core_axis_name="c", subcore_axis_name="s",
                                  num_cores=_NC, num_subcores=_NS)

    @functools.partial(
        pl.kernel, mesh=mesh,
        out_type=jax.ShapeDtypeStruct((2, N, W), jnp.float32),
        scratch_types=[
            pltpu.VMEM((_CH,), jnp.int32),
            pltpu.VMEM((_CH, W), jnp.float32),
            pltpu.VMEM((zch, W), jnp.float32),
            pltpu.VMEM_SHARED((N, W), jnp.float32),
            pltpu.SemaphoreType.DMA,
        ],
    )
    def k(val_hbm, idx_hbm, out_hbm, idx_v, rows_v, stage_v, acc_sh, sem):
        c = lax.axis_index("c")
        s = lax.axis_index("s")

        def zrow(i, _):
            def zcol(j, _):
                stage_v[i, pl.ds(j * 16, 16)] = jnp.zeros((16,), jnp.float32)
                return 0
            lax.fori_loop(0, W // 16, zcol, 0)
            return 0
        lax.fori_loop(0, zch, zrow, 0)

        def zdma(k2, _):
            t = k2 * _NS + s

            @pl.when(t < nz)
            def _():
                pltpu.sync_copy(stage_v, acc_sh.at[pl.ds(t * zch, zch)])
            return 0
        lax.fori_loop(0, per_sz, zdma, 0)
        plsc.subcore_barrier()

        def body(i, _):
            kk = i * _NS + s
            j = kk * _NC + c

            @pl.when(j < nch)
            def _():
                base = j * _CH
                pltpu.sync_copy(idx_hbm.at[pl.ds(base, _CH)], idx_v)
                pltpu.sync_copy(val_hbm.at[pl.ds(base, _CH)], rows_v)
                pltpu.sync_copy(rows_v, acc_sh.at[idx_v], add=True)
            return 0

        lax.fori_loop(0, per_s, body, 0)
        plsc.subcore_barrier()

        def wb(k2, _):
            t = k2 * _NS + s

            @pl.when(t < nz)
            def _():
                r0 = t * zch
                pltpu.sync_copy(acc_sh.at[pl.ds(r0, zch)], stage_v)

                @pl.when(c == 0)
                def _():
                    pltpu.sync_copy(stage_v, out_hbm.at[0, pl.ds(r0, zch)])

                @pl.when(c == 1)
                def _():
                    pltpu.sync_copy(stage_v, out_hbm.at[1, pl.ds(r0, zch)])
            return 0

        lax.fori_loop(0, per_sz, wb, 0)

    return k


def _scatter_pad_call(val, idx, N):
    E2, W = val.shape
    parts = _sc_scatter_pad_fn(N, E2, W)(val, idx)
    return parts[0] + parts[1]


def _make_edge_diff(N, E):
    @jax.custom_vjp
    def edge_diff(pos, period_vec, src, dst):
        return pos[dst] - pos[src] + period_vec

    def ed_fwd(pos, period_vec, src, dst):
        return pos[dst] - pos[src] + period_vec, (src, dst)

    def ed_bwd(res, g):
        src, dst = res
        gp = jnp.pad(g, ((0, 0), (0, 125)))
        val = jnp.concatenate([gp, -gp], axis=0)
        idx = jnp.concatenate([dst, src], axis=0)
        d_pos = _scatter_pad_call(val, idx, N)[:, :3]
        return d_pos, g, _f0(src), _f0(dst)

    edge_diff.defvjp(ed_fwd, ed_bwd)
    return edge_diff


def _vmul_rows(dst_ref, a_ref, b_ref, rows, cols):
    def row(e, _):
        for jj in range(cols // 16):
            sl = pl.ds(jj * 16, 16)
            dst_ref[e, sl] = a_ref[e, sl] * b_ref[e, sl]
        return 0
    lax.fori_loop(0, rows, row, 0)


@functools.lru_cache(maxsize=None)
def _sc_conv_fwd_fn(N, H, E):
    Hc = H // _NC
    CHF = 64
    nch = E // CHF
    assert nch * CHF == E
    per_s = -(-nch // _NS)
    zch = 40
    nz = N // zch
    assert nz * zch == N
    per_sz = -(-nz // _NS)
    mesh = plsc.VectorSubcoreMesh(core_axis_name="c", subcore_axis_name="s",
                                  num_cores=_NC, num_subcores=_NS)

    @functools.partial(
        pl.kernel, mesh=mesh,
        out_type=jax.ShapeDtypeStruct((N, H), jnp.float32),
        scratch_types=[
            pltpu.VMEM((CHF,), jnp.int32),
            pltpu.VMEM((CHF,), jnp.int32),
            pltpu.VMEM((CHF,), jnp.int32),
            pltpu.VMEM((CHF,), jnp.int32),
            pltpu.VMEM((CHF, Hc), jnp.float32),
            pltpu.VMEM((CHF, Hc), jnp.float32),
            pltpu.VMEM((CHF, Hc), jnp.float32),
            pltpu.VMEM((CHF, Hc), jnp.float32),
            pltpu.VMEM((zch, Hc), jnp.float32),
            pltpu.VMEM_SHARED((N, Hc), jnp.float32),
            pltpu.SemaphoreType.DMA,
            pltpu.SemaphoreType.DMA,
            pltpu.SemaphoreType.DMA,
            pltpu.SemaphoreType.DMA,
        ],
    )
    def k(h0_hbm, h1_hbm, mod_hbm, src_hbm, dst_hbm, out_hbm,
          si0, di0, si1, di1, g0, m0, g1, m1, stage_v, acc_sh,
          sg0, sm0, sg1, sm1):
        c = lax.axis_index("c")
        s = lax.axis_index("s")
        bufs = ((si0, di0, g0, m0, sg0, sm0), (si1, di1, g1, m1, sg1, sm1))

        def zrow(i, _):
            def zcol(j, _):
                stage_v[i, pl.ds(j * 16, 16)] = jnp.zeros((16,), jnp.float32)
                return 0
            lax.fori_loop(0, Hc // 16, zcol, 0)
            return 0
        lax.fori_loop(0, zch, zrow, 0)

        def zdma(k2, _):
            t = k2 * _NS + s

            @pl.when(t < nz)
            def _():
                pltpu.sync_copy(stage_v, acc_sh.at[pl.ds(t * zch, zch)])
            return 0
        lax.fori_loop(0, per_sz, zdma, 0)
        plsc.subcore_barrier()

        def prefetch(i, b):
            si, di, g, m, sg, sm = bufs[b]
            j = i * _NS + s

            @pl.when(j < nch)
            def _():
                base = j * CHF
                pltpu.sync_copy(src_hbm.at[pl.ds(base, CHF)], si)
                pltpu.sync_copy(dst_hbm.at[pl.ds(base, CHF)], di)

                @pl.when(c == 0)
                def _():
                    pltpu.async_copy(h0_hbm.at[si], g, sg)
                    pltpu.async_copy(
                        mod_hbm.at[pl.ds(base, CHF), pl.ds(0, Hc)], m, sm)

                @pl.when(c == 1)
                def _():
                    pltpu.async_copy(h1_hbm.at[si], g, sg)
                    pltpu.async_copy(
                        mod_hbm.at[pl.ds(base, CHF), pl.ds(Hc, Hc)], m, sm)

        def process(i, b):
            si, di, g, m, sg, sm = bufs[b]
            j = i * _NS + s

            @pl.when(j < nch)
            def _():
                base = j * CHF

                @pl.when(c == 0)
                def _():
                    pltpu.make_async_copy(h0_hbm.at[si], g, sg).wait()
                    pltpu.make_async_copy(
                        mod_hbm.at[pl.ds(base, CHF), pl.ds(0, Hc)], m, sm).wait()

                @pl.when(c == 1)
                def _():
                    pltpu.make_async_copy(h1_hbm.at[si], g, sg).wait()
                    pltpu.make_async_copy(
                        mod_hbm.at[pl.ds(base, CHF), pl.ds(Hc, Hc)], m, sm).wait()

                _vmul_rows(m, m, g, CHF, Hc)
                pltpu.sync_copy(m, acc_sh.at[di], add=True)

        prefetch(0, 0)

        def body(t, _):
            i0 = 2 * t
            prefetch(i0 + 1, 1)
            process(i0, 0)
            prefetch(i0 + 2, 0)
            process(i0 + 1, 1)
            return 0

        lax.fori_loop(0, (per_s + 1) // 2, body, 0)
        plsc.subcore_barrier()

        def wb(k2, _):
            t = k2 * _NS + s

            @pl.when(t < nz)
            def _():
                r0 = t * zch
                pltpu.sync_copy(acc_sh.at[pl.ds(r0, zch)], stage_v)

                @pl.when(c == 0)
                def _():
                    pltpu.sync_copy(stage_v,
                                    out_hbm.at[pl.ds(r0, zch), pl.ds(0, Hc)])

                @pl.when(c == 1)
                def _():
                    pltpu.sync_copy(stage_v,
                                    out_hbm.at[pl.ds(r0, zch), pl.ds(Hc, Hc)])
            return 0

        lax.fori_loop(0, per_sz, wb, 0)

    return k


@functools.lru_cache(maxsize=None)
def _sc_conv_bwd_fn(N, H, E):
    Hc = H // _NC
    nch = E // _CH
    assert nch * _CH == E
    per_s = -(-nch // _NS)
    CHB = 64
    nchb = E // CHB
    per_sb = -(-nchb // _NS)
    zch = 40
    nz = N // zch
    assert nz * zch == N
    per_sz = -(-nz // _NS)
    mesh = plsc.VectorSubcoreMesh(core_axis_name="c", subcore_axis_name="s",
                                  num_cores=_NC, num_subcores=_NS)

    @functools.partial(
        pl.kernel, mesh=mesh,
        out_type=jax.ShapeDtypeStruct((E, H), jnp.float32),
        scratch_types=[
            pltpu.VMEM((_CH,), jnp.int32),
            pltpu.VMEM((_CH,), jnp.int32),
            pltpu.VMEM((_CH,), jnp.int32),
            pltpu.VMEM((_CH,), jnp.int32),
            pltpu.VMEM((_CH, Hc), jnp.float32),
            pltpu.VMEM((_CH, Hc), jnp.float32),
            pltpu.VMEM((_CH, Hc), jnp.float32),
            pltpu.VMEM((_CH, Hc), jnp.float32),
            pltpu.SemaphoreType.DMA,
            pltpu.SemaphoreType.DMA,
            pltpu.SemaphoreType.DMA,
            pltpu.SemaphoreType.DMA,
        ],
    )
    def k_dmod(da0_hbm, da1_hbm, h0_hbm, h1_hbm, src_hbm, dst_hbm,
               dmod_hbm, si0, di0, si1, di1, t0, g0, t1, g1,
               st0, sg0, st1, sg1):
        c = lax.axis_index("c")
        s = lax.axis_index("s")
        bufs = ((si0, di0, t0, g0, st0, sg0), (si1, di1, t1, g1, st1, sg1))

        def prefetch(i, b):
            si, di, t, g, st, sg = bufs[b]
            j = i * _NS + s

            @pl.when(j < nch)
            def _():
                base = j * _CH
                pltpu.sync_copy(src_hbm.at[pl.ds(base, _CH)], si)
                pltpu.sync_copy(dst_hbm.at[pl.ds(base, _CH)], di)

                @pl.when(c == 0)
                def _():
                    pltpu.async_copy(da0_hbm.at[di], t, st)
                    pltpu.async_copy(h0_hbm.at[si], g, sg)

                @pl.when(c == 1)
                def _():
                    pltpu.async_copy(da1_hbm.at[di], t, st)
                    pltpu.async_copy(h1_hbm.at[si], g, sg)

        def process(i, b):
            si, di, t, g, st, sg = bufs[b]
            j = i * _NS + s

            @pl.when(j < nch)
            def _():
                base = j * _CH

                @pl.when(c == 0)
                def _():
                    pltpu.make_async_copy(da0_hbm.at[di], t, st).wait()
                    pltpu.make_async_copy(h0_hbm.at[si], g, sg).wait()
                    _vmul_rows(g, g, t, _CH, Hc)
                    pltpu.sync_copy(
                        g, dmod_hbm.at[pl.ds(base, _CH), pl.ds(0, Hc)])

                @pl.when(c == 1)
                def _():
                    pltpu.make_async_copy(da1_hbm.at[di], t, st).wait()
                    pltpu.make_async_copy(h1_hbm.at[si], g, sg).wait()
                    _vmul_rows(g, g, t, _CH, Hc)
                    pltpu.sync_copy(
                        g, dmod_hbm.at[pl.ds(base, _CH), pl.ds(Hc, Hc)])

        prefetch(0, 0)

        def body(tt, _):
            i0 = 2 * tt
            prefetch(i0 + 1, 1)
            process(i0, 0)
            prefetch(i0 + 2, 0)
            process(i0 + 1, 1)
            return 0

        lax.fori_loop(0, (per_s + 1) // 2, body, 0)

    @functools.partial(
        pl.kernel, mesh=mesh,
        out_type=jax.ShapeDtypeStruct((N, H), jnp.float32),
        scratch_types=[
            pltpu.VMEM((CHB,), jnp.int32),
            pltpu.VMEM((CHB,), jnp.int32),
            pltpu.VMEM((CHB,), jnp.int32),
            pltpu.VMEM((CHB,), jnp.int32),
            pltpu.VMEM((CHB, Hc), jnp.float32),
            pltpu.VMEM((CHB, Hc), jnp.float32),
            pltpu.VMEM((CHB, Hc), jnp.float32),
            pltpu.VMEM((CHB, Hc), jnp.float32),
            pltpu.VMEM((zch, Hc), jnp.float32),
            pltpu.VMEM_SHARED((N, Hc), jnp.float32),
            pltpu.SemaphoreType.DMA,
            pltpu.SemaphoreType.DMA,
            pltpu.SemaphoreType.DMA,
            pltpu.SemaphoreType.DMA,
        ],
    )
    def k_dh(da0_hbm, da1_hbm, mod_hbm, src_hbm, dst_hbm, dh_hbm,
             si0, di0, si1, di1, t0, m0, t1, m1, stage_v, acc_sh,
             st0, sm0, st1, sm1):
        c = lax.axis_index("c")
        s = lax.axis_index("s")
        bufs = ((si0, di0, t0, m0, st0, sm0), (si1, di1, t1, m1, st1, sm1))

        def zrow(i, _):
            def zcol(j, _):
                stage_v[i, pl.ds(j * 16, 16)] = jnp.zeros((16,), jnp.float32)
                return 0
            lax.fori_loop(0, Hc // 16, zcol, 0)
            return 0
        lax.fori_loop(0, zch, zrow, 0)

        def zdma(k2, _):
            t = k2 * _NS + s

            @pl.when(t < nz)
            def _():
                pltpu.sync_copy(stage_v, acc_sh.at[pl.ds(t * zch, zch)])
            return 0
        lax.fori_loop(0, per_sz, zdma, 0)
        plsc.subcore_barrier()

        def prefetch(i, b):
            si, di, t, m, st, sm = bufs[b]
            j = i * _NS + s

            @pl.when(j < nchb)
            def _():
                base = j * CHB
                pltpu.sync_copy(src_hbm.at[pl.ds(base, CHB)], si)
                pltpu.sync_copy(dst_hbm.at[pl.ds(base, CHB)], di)

                @pl.when(c == 0)
                def _():
                    pltpu.async_copy(da0_hbm.at[di], t, st)
                    pltpu.async_copy(
                        mod_hbm.at[pl.ds(base, CHB), pl.ds(0, Hc)], m, sm)

                @pl.when(c == 1)
                def _():
                    pltpu.async_copy(da1_hbm.at[di], t, st)
                    pltpu.async_copy(
                        mod_hbm.at[pl.ds(base, CHB), pl.ds(Hc, Hc)], m, sm)

        def process(i, b):
            si, di, t, m, st, sm = bufs[b]
            j = i * _NS + s

            @pl.when(j < nchb)
            def _():
                base = j * CHB

                @pl.when(c == 0)
                def _():
                    pltpu.make_async_copy(da0_hbm.at[di], t, st).wait()
                    pltpu.make_async_copy(
                        mod_hbm.at[pl.ds(base, CHB), pl.ds(0, Hc)], m, sm).wait()

                @pl.when(c == 1)
                def _():
                    pltpu.make_async_copy(da1_hbm.at[di], t, st).wait()
                    pltpu.make_async_copy(
                        mod_hbm.at[pl.ds(base, CHB), pl.ds(Hc, Hc)], m, sm).wait()

                _vmul_rows(m, m, t, CHB, Hc)
                pltpu.sync_copy(m, acc_sh.at[si], add=True)

        prefetch(0, 0)

        def body(tt, _):
            i0 = 2 * tt
            prefetch(i0 + 1, 1)
            process(i0, 0)
            prefetch(i0 + 2, 0)
            process(i0 + 1, 1)
            return 0

        lax.fori_loop(0, (per_sb + 1) // 2, body, 0)
        plsc.subcore_barrier()

        def wb(k2, _):
            t = k2 * _NS + s

            @pl.when(t < nz)
            def _():
                r0 = t * zch
                pltpu.sync_copy(acc_sh.at[pl.ds(r0, zch)], stage_v)

                @pl.when(c == 0)
                def _():
                    pltpu.sync_copy(stage_v,
                                    dh_hbm.at[pl.ds(r0, zch), pl.ds(0, Hc)])

                @pl.when(c == 1)
                def _():
                    pltpu.sync_copy(stage_v,
                                    dh_hbm.at[pl.ds(r0, zch), pl.ds(Hc, Hc)])
            return 0

        lax.fori_loop(0, per_sz, wb, 0)

    return k_dmod, k_dh


def _conv_fwd_call(h, mod, src, dst):
    N, H = h.shape
    Hc = H // _NC
    E = src.shape[0]
    return _sc_conv_fwd_fn(N, H, E)(h[:, :Hc], h[:, Hc:], mod, src, dst)


def _conv_bwd_call(g, h, mod, src, dst):
    N, H = h.shape
    Hc = H // _NC
    E = src.shape[0]
    k_dmod, k_dh = _sc_conv_bwd_fn(N, H, E)
    g0, g1 = g[:, :Hc], g[:, Hc:]
    d_mod = k_dmod(g0, g1, h[:, :Hc], h[:, Hc:], src, dst)
    d_h = k_dh(g0, g1, mod, src, dst)
    return d_mod, d_h


def _dmod_call(g, h, src, dst):
    N, H = h.shape
    Hc = H // _NC
    E = src.shape[0]
    k_dmod, _ = _sc_conv_bwd_fn(N, H, E)
    return k_dmod(g[:, :Hc], g[:, Hc:], h[:, :Hc], h[:, Hc:], src, dst)


def _make_conv_op(need_dh):
    @jax.custom_vjp
    def conv_agg(h, mod, src, dst):
        return _conv_fwd_call(h, mod, src, dst)

    def cv_fwd(h, mod, src, dst):
        return _conv_fwd_call(h, mod, src, dst), (h, mod, src, dst)

    def cv_bwd(res, g):
        h, mod, src, dst = res
        if need_dh:
            d_mod, d_h = _conv_bwd_call(g, h, mod, src, dst)
        else:
            d_mod = _dmod_call(g, h, src, dst)
            d_h = jnp.zeros_like(h)
        return d_h, d_mod, _f0(src), _f0(dst)

    conv_agg.defvjp(cv_fwd, cv_bwd)
    return conv_agg


def _gather_call(table, idx):
    N, H = table.shape
    (E,) = idx.shape
    return _sc_gather_fn(N, H, E)(table, idx)


def _scatter_call(msg, idx, N):
    E, H = msg.shape
    return _sc_scatter_fn(N, H, E)(msg, idx)


def _f0(idx):
    return np.zeros(idx.shape, jax.dtypes.float0)


def _make_sc_ops(N):
    @jax.custom_vjp
    def gather_rows(table, idx):
        return _gather_call(table, idx)

    def gather_fwd(table, idx):
        return _gather_call(table, idx), idx

    def gather_bwd(idx, g):
        return _scatter_call(g, idx, N), _f0(idx)

    gather_rows.defvjp(gather_fwd, gather_bwd)

    @jax.custom_vjp
    def seg_sum(msg, idx):
        return _scatter_call(msg, idx, N)

    def seg_fwd(msg, idx):
        return _scatter_call(msg, idx, N), idx

    def seg_bwd(idx, g):
        return _gather_call(g, idx), _f0(idx)

    seg_sum.defvjp(seg_fwd, seg_bwd)
    return gather_rows, seg_sum


def _mm(a, b):
    return jnp.dot(a, b, preferred_element_type=jnp.float32)


@functools.lru_cache(maxsize=None)
def _edge_mod_fwd_fn(E, H, K1, K2):
    def body(rbf_ref, shp_ref, wr_ref, wsp_ref, mod_ref):
        radial = _mm(rbf_ref[...], wr_ref[...])
        shw = _mm(shp_ref[...], wsp_ref[...])
        mod_ref[...] = radial * shw

    return pl.pallas_call(
        body,
        grid=(E // _BE,),
        in_specs=[
            pl.BlockSpec((_BE, K1), lambda i: (i, 0)),
            pl.BlockSpec((_BE, K2), lambda i: (i, 0)),
            pl.BlockSpec((K1, H), lambda i: (0, 0)),
            pl.BlockSpec((K2, H), lambda i: (0, 0)),
        ],
        out_specs=pl.BlockSpec((_BE, H), lambda i: (i, 0)),
        out_shape=jax.ShapeDtypeStruct((E, H), jnp.float32),
    )


@functools.lru_cache(maxsize=None)
def _edge_mod_bwd_fn(E, H, K1, K2):
    def body(g_ref, rbf_ref, shp_ref, wr_ref, wsp_ref, wrt_ref, wspt_ref,
             drbf_ref, dshp_ref):
        g = g_ref[...]
        radial = _mm(rbf_ref[...], wr_ref[...])
        shw = _mm(shp_ref[...], wsp_ref[...])
        drbf_ref[...] = _mm(g * shw, wrt_ref[...])
        dshp_ref[...] = _mm(g * radial, wspt_ref[...])

    return pl.pallas_call(
        body,
        grid=(E // _BE,),
        in_specs=[
            pl.BlockSpec((_BE, H), lambda i: (i, 0)),
            pl.BlockSpec((_BE, K1), lambda i: (i, 0)),
            pl.BlockSpec((_BE, K2), lambda i: (i, 0)),
            pl.BlockSpec((K1, H), lambda i: (0, 0)),
            pl.BlockSpec((K2, H), lambda i: (0, 0)),
            pl.BlockSpec((H, K1), lambda i: (0, 0)),
            pl.BlockSpec((H, K2), lambda i: (0, 0)),
        ],
        out_specs=[
            pl.BlockSpec((_BE, K1), lambda i: (i, 0)),
            pl.BlockSpec((_BE, K2), lambda i: (i, 0)),
        ],
        out_shape=[
            jax.ShapeDtypeStruct((E, K1), jnp.float32),
            jax.ShapeDtypeStruct((E, K2), jnp.float32),
        ],
    )


def _edge_mod_fwd_call(rbf, shp, wr, wsp):
    E, K1 = rbf.shape
    K2 = shp.shape[1]
    H = wr.shape[1]
    return _edge_mod_fwd_fn(E, H, K1, K2)(rbf, shp, wr, wsp)


def _edge_mod_bwd_call(g, rbf, shp, wr, wsp):
    E, K1 = rbf.shape
    K2 = shp.shape[1]
    H = wr.shape[1]
    return _edge_mod_bwd_fn(E, H, K1, K2)(g, rbf, shp, wr, wsp, wr.T, wsp.T)


@jax.custom_vjp
def _edge_mod(rbf, shp, wr, wsp):
    return _edge_mod_fwd_call(rbf, shp, wr, wsp)


def _edge_mod_f(rbf, shp, wr, wsp):
    return _edge_mod_fwd_call(rbf, shp, wr, wsp), (rbf, shp, wr, wsp)


def _edge_mod_b(res, g):
    rbf, shp, wr, wsp = res
    drbf, dshp = _edge_mod_bwd_call(g, rbf, shp, wr, wsp)
    return drbf, dshp, jnp.zeros_like(wr), jnp.zeros_like(wsp)


_edge_mod.defvjp(_edge_mod_f, _edge_mod_b)


@functools.lru_cache(maxsize=None)
def _node_upd_fwd_fn(N, H, A):
    def body(h_ref, agg_ref, xa_ref, ws_ref, wu_ref, wa_ref, pre_ref, hn_ref):
        pre = (_mm(h_ref[...], ws_ref[...]) + _mm(agg_ref[...], wu_ref[...])
               + _mm(xa_ref[...], wa_ref[...]))
        pre_ref[...] = pre
        hn_ref[...] = pre * jax.nn.sigmoid(pre)

    return pl.pallas_call(
        body,
        grid=(N // _BN,),
        in_specs=[
            pl.BlockSpec((_BN, H), lambda i: (i, 0)),
            pl.BlockSpec((_BN, H), lambda i: (i, 0)),
            pl.BlockSpec((_BN, A), lambda i: (i, 0)),
            pl.BlockSpec((H, H), lambda i: (0, 0)),
            pl.BlockSpec((H, H), lambda i: (0, 0)),
            pl.BlockSpec((A, H), lambda i: (0, 0)),
        ],
        out_specs=[
            pl.BlockSpec((_BN, H), lambda i: (i, 0)),
            pl.BlockSpec((_BN, H), lambda i: (i, 0)),
        ],
        out_shape=[
            jax.ShapeDtypeStruct((N, H), jnp.float32),
            jax.ShapeDtypeStruct((N, H), jnp.float32),
        ],
    )


@functools.lru_cache(maxsize=None)
def _node_upd_bwd_fn(N, H):
    def body(g_ref, pre_ref, wst_ref, wut_ref, dh_ref, dagg_ref):
        gp = g_ref[...] * _dsilu(pre_ref[...])
        dh_ref[...] = _mm(gp, wst_ref[...])
        dagg_ref[...] = _mm(gp, wut_ref[...])

    return pl.pallas_call(
        body,
        grid=(N // _BN,),
        in_specs=[
            pl.BlockSpec((_BN, H), lambda i: (i, 0)),
            pl.BlockSpec((_BN, H), lambda i: (i, 0)),
            pl.BlockSpec((H, H), lambda i: (0, 0)),
            pl.BlockSpec((H, H), lambda i: (0, 0)),
        ],
        out_specs=[
            pl.BlockSpec((_BN, H), lambda i: (i, 0)),
            pl.BlockSpec((_BN, H), lambda i: (i, 0)),
        ],
        out_shape=[
            jax.ShapeDtypeStruct((N, H), jnp.float32),
            jax.ShapeDtypeStruct((N, H), jnp.float32),
        ],
    )


def _node_upd_fwd_call(h, agg, xa, ws, wu, wa):
    N, H = h.shape
    A = xa.shape[1]
    return _node_upd_fwd_fn(N, H, A)(h, agg, xa, ws, wu, wa)


def _node_upd_bwd_call(g, pre, ws, wu):
    N, H = g.shape
    return _node_upd_bwd_fn(N, H)(g, pre, ws.T, wu.T)


@jax.custom_vjp
def _node_update(h, agg, xa, ws, wu, wa):
    _, hn = _node_upd_fwd_call(h, agg, xa, ws, wu, wa)
    return hn


def _node_update_f(h, agg, xa, ws, wu, wa):
    pre, hn = _node_upd_fwd_call(h, agg, xa, ws, wu, wa)
    return hn, (pre, ws, wu, xa.shape[1])


def _node_update_b(res, g):
    pre, ws, wu, A = res
    dh, dagg = _node_upd_bwd_call(g, pre, ws, wu)
    return (dh, dagg, jnp.zeros((g.shape[0], A), jnp.float32),
            jnp.zeros_like(ws), jnp.zeros_like(wu),
            jnp.zeros((A, g.shape[1]), jnp.float32))


_node_update.defvjp(_node_update_f, _node_update_b)


@functools.lru_cache(maxsize=None)
def _head_fwd_fn(N, H, P):
    def body(h_ref, w_ref, b_ref, z_ref, hp_ref):
        z = _mm(h_ref[...], w_ref[...]) + b_ref[...]
        z_ref[...] = z
        hp_ref[...] = z * jax.nn.sigmoid(z)

    return pl.pallas_call(
        body,
        grid=(N // _BN,),
        in_specs=[
            pl.BlockSpec((_BN, H), lambda i: (i, 0)),
            pl.BlockSpec((H, P), lambda i: (0, 0)),
            pl.BlockSpec((1, P), lambda i: (0, 0)),
        ],
        out_specs=[
            pl.BlockSpec((_BN, P), lambda i: (i, 0)),
            pl.BlockSpec((_BN, P), lambda i: (i, 0)),
        ],
        out_shape=[
            jax.ShapeDtypeStruct((N, P), jnp.float32),
            jax.ShapeDtypeStruct((N, P), jnp.float32),
        ],
    )


@functools.lru_cache(maxsize=None)
def _head_bwd_fn(N, H, P):
    def body(g_ref, z_ref, wt_ref, dh_ref):
        dh_ref[...] = _mm(g_ref[...] * _dsilu(z_ref[...]), wt_ref[...])

    return pl.pallas_call(
        body,
        grid=(N // _BN,),
        in_specs=[
            pl.BlockSpec((_BN, P), lambda i: (i, 0)),
            pl.BlockSpec((_BN, P), lambda i: (i, 0)),
            pl.BlockSpec((P, H), lambda i: (0, 0)),
        ],
        out_specs=pl.BlockSpec((_BN, H), lambda i: (i, 0)),
        out_shape=jax.ShapeDtypeStruct((N, H), jnp.float32),
    )


def _head_fwd_call(h, w, b):
    N, H = h.shape
    P = w.shape[1]
    return _head_fwd_fn(N, H, P)(h, w, b.reshape(1, P))


def _head_bwd_call(g, z, w):
    N, P = g.shape
    H = w.shape[0]
    return _head_bwd_fn(N, H, P)(g, z, w.T)


@jax.custom_vjp
def _head(h, w, b):
    _, hp = _head_fwd_call(h, w, b)
    return hp


def _head_f(h, w, b):
    z, hp = _head_fwd_call(h, w, b)
    return hp, (z, w)


def _head_b(res, g):
    z, w = res
    return _head_bwd_call(g, z, w), jnp.zeros_like(w), jnp.zeros((w.shape[1],), jnp.float32)


_head.defvjp(_head_f, _head_b)


@functools.lru_cache(maxsize=None)
def _embed_fn(N, A, H):
    def body(xa_ref, w_ref, b_ref, out_ref):
        out_ref[...] = _mm(xa_ref[...], w_ref[...]) + b_ref[...]

    return pl.pallas_call(
        body,
        grid=(N // _BN,),
        in_specs=[
            pl.BlockSpec((_BN, A), lambda i: (i, 0)),
            pl.BlockSpec((A, H), lambda i: (0, 0)),
            pl.BlockSpec((1, H), lambda i: (0, 0)),
        ],
        out_specs=pl.BlockSpec((_BN, H), lambda i: (i, 0)),
        out_shape=jax.ShapeDtypeStruct((N, H), jnp.float32),
    )


def _embed_call(xa, w, b):
    N, A = xa.shape
    H = w.shape[1]
    return _embed_fn(N, A, H)(xa, w, b.reshape(1, H))


def kernel(x, pos, edge_index, period_vec, batch, per_config_dataset_idx,
           elem_table, W_embed, b_embed, W_rbf, W_sh, W_self, W_upd, W_attr,
           W_p1, b_p1, W_p2, b_p2, scale, shift):
    N = pos.shape[0]
    E = edge_index.shape[1]
    H = W_embed.shape[1]
    G = per_config_dataset_idx.shape[0]
    nlayers, nrbf, _ = W_rbf.shape
    sh_dim = W_sh.shape[1]
    cutoff = 6.0

    src = edge_index[0]
    dst = edge_index[1]
    x_attr = elem_table[x]
    h0 = _embed_call(x_attr, W_embed, b_embed)
    k2 = 16
    W_shp = jnp.concatenate(
        [W_sh, jnp.zeros((nlayers, k2 - sh_dim, H), jnp.float32)], axis=1)

    conv_agg = _make_conv_op(True)
    conv_agg0 = _make_conv_op(False)
    edge_diff = _make_edge_diff(N, E)

    centers = jnp.linspace(0.0, cutoff, nrbf)
    width = cutoff / nrbf
    pool = (batch[:, None] == jnp.arange(G)[None, :]).astype(jnp.float32)

    def efn(pos_in):
        edge_vec = edge_diff(pos_in, period_vec, src, dst)
        lengths = jnp.sqrt(jnp.sum(edge_vec * edge_vec, axis=-1) + 1e-12)
        unit = edge_vec / lengths[:, None]
        ex, ey, ez = unit[:, 0], unit[:, 1], unit[:, 2]
        zero = jnp.zeros_like(ex)
        shp = jnp.stack(
            [jnp.ones_like(ex), ex, ey, ez, ex * ey, ey * ez, ez * ex,
             ex * ex - ey * ey, 3.0 * ez * ez - 1.0,
             zero, zero, zero, zero, zero, zero, zero], axis=-1)
        rbf = jnp.exp(-jnp.square((lengths[:, None] - centers[None, :]) / width))
        env = 0.5 * (jnp.cos(jnp.pi * jnp.clip(lengths / cutoff, 0.0, 1.0)) + 1.0)
        rbf = rbf * env[:, None]

        h = h0
        for l in range(nlayers):
            mod = _edge_mod(rbf, shp, W_rbf[l], W_shp[l])
            agg = (conv_agg if l > 0 else conv_agg0)(h, mod, src, dst)
            h = _node_update(h, agg, x_attr, W_self[l], W_upd[l], W_attr[l])

        hp1 = _head(h, W_p1, b_p1)
        hp2 = _mm(hp1, W_p2) + b_p2
        graph_e = _mm(pool.T, hp2)
        energies_all = graph_e * scale + shift
        return energies_all[jnp.arange(G), per_config_dataset_idx]

    energies, vjp_fn = jax.vjp(efn, pos)
    forces = -vjp_fn(jnp.ones_like(energies))[0]
    return (energies, forces)
